# speedup vs baseline: 1.1142x; 1.1142x over previous
import functools
import math

import jax
import jax.numpy as jnp
from jax.experimental import pallas as pl

N = 10000
E = 320000
HID = 128
EDIM = 16
TDIM = 64

BE = 3200
BN = 1000


def _silu(v):
    return v * jax.nn.sigmoid(v)


def _edge_body(ein_ref, xd_ref, ea_ref, w1t_ref, w2_ref, b2_ref, attw_ref,
               attb_ref, cw1_ref, cb1_ref, cw2_ref, m_ref, cdw_ref):
    xd = xd_ref[...]
    radial = jnp.sum(xd * xd, axis=1, keepdims=True)
    ra = jnp.concatenate([radial, ea_ref[...]], axis=1)
    t1 = ein_ref[...] + jnp.dot(ra, w1t_ref[...],
                                preferred_element_type=jnp.float32)
    m1 = _silu(t1)
    m2 = _silu(jnp.dot(m1, w2_ref[...],
                       preferred_element_type=jnp.float32) + b2_ref[...])
    att = jax.nn.sigmoid(jnp.dot(m2, attw_ref[...],
                                 preferred_element_type=jnp.float32)
                         + attb_ref[0, 0])
    m = m2 * att
    c1 = _silu(jnp.dot(m, cw1_ref[...],
                       preferred_element_type=jnp.float32) + cb1_ref[...])
    cw = jnp.dot(c1, cw2_ref[...], preferred_element_type=jnp.float32)
    m_ref[...] = m
    cdw_ref[...] = xd * (cw / jnp.sqrt(radial + 1e-8))


def _edge_mlp(ein_pre, xdiff, eattr, w1t, w2, b2, attw, attb, cw1, cb1, cw2):
    grid = (E // BE,)
    eb = lambda i: (i, 0)
    wb = lambda i: (0, 0)
    return pl.pallas_call(
        _edge_body,
        grid=grid,
        in_specs=[
            pl.BlockSpec((BE, HID), eb),
            pl.BlockSpec((BE, 3), eb),
            pl.BlockSpec((BE, EDIM), eb),
            pl.BlockSpec((1 + EDIM, HID), wb),
            pl.BlockSpec((HID, HID), wb),
            pl.BlockSpec((1, HID), wb),
            pl.BlockSpec((HID, 1), wb),
            pl.BlockSpec((1, 1), wb),
            pl.BlockSpec((HID, HID), wb),
            pl.BlockSpec((1, HID), wb),
            pl.BlockSpec((HID, 1), wb),
        ],
        out_specs=[
            pl.BlockSpec((BE, HID), eb),
            pl.BlockSpec((BE, 3), eb),
        ],
        out_shape=[
            jax.ShapeDtypeStruct((E, HID), jnp.float32),
            jax.ShapeDtypeStruct((E, 3), jnp.float32),
        ],
    )(ein_pre, xdiff, eattr, w1t, w2, b2, attw, attb, cw1, cb1, cw2)


def _node_body(h_ref, mi_ref, w1h_ref, w1m_ref, nb1_ref, w2_ref, nb2_ref,
               g_ref, b_ref, ew1a_ref, ew1b_ref, eb1_ref,
               h_out_ref, a_ref, b_out_ref):
    h = h_ref[...]
    z = (jnp.dot(h, w1h_ref[...], preferred_element_type=jnp.float32)
         + jnp.dot(mi_ref[...], w1m_ref[...],
                   preferred_element_type=jnp.float32) + nb1_ref[...])
    dh = jnp.dot(_silu(z), w2_ref[...],
                 preferred_element_type=jnp.float32) + nb2_ref[...]
    hn = h + dh
    mu = jnp.mean(hn, axis=1, keepdims=True)
    var = jnp.mean((hn - mu) ** 2, axis=1, keepdims=True)
    hln = (hn - mu) / jnp.sqrt(var + 1e-5) * g_ref[...] + b_ref[...]
    h_out_ref[...] = hln
    a_ref[...] = jnp.dot(hln, ew1a_ref[...],
                         preferred_element_type=jnp.float32) + eb1_ref[...]
    b_out_ref[...] = jnp.dot(hln, ew1b_ref[...],
                             preferred_element_type=jnp.float32)


def _node_update(h, m_i, w1h, w1m, nb1, w2, nb2, g, b, ew1a, ew1b, eb1):
    grid = (N // BN,)
    nb = lambda i: (i, 0)
    wb = lambda i: (0, 0)
    return pl.pallas_call(
        _node_body,
        grid=grid,
        in_specs=[
            pl.BlockSpec((BN, HID), nb),
            pl.BlockSpec((BN, HID), nb),
            pl.BlockSpec((HID, HID), wb),
            pl.BlockSpec((HID, HID), wb),
            pl.BlockSpec((1, HID), wb),
            pl.BlockSpec((HID, HID), wb),
            pl.BlockSpec((1, HID), wb),
            pl.BlockSpec((1, HID), wb),
            pl.BlockSpec((1, HID), wb),
            pl.BlockSpec((HID, HID), wb),
            pl.BlockSpec((HID, HID), wb),
            pl.BlockSpec((1, HID), wb),
        ],
        out_specs=[pl.BlockSpec((BN, HID), nb)] * 3,
        out_shape=[jax.ShapeDtypeStruct((N, HID), jnp.float32)] * 3,
    )(h, m_i, w1h, w1m, nb1, w2, nb2, g, b, ew1a, ew1b, eb1)


def _pro_body(h_ref, ew_ref, te_ref, ew1a_ref, ew1b_ref, eb1_ref,
              h_out_ref, a_ref, b_out_ref):
    h0 = (jnp.dot(h_ref[...], ew_ref[...], preferred_element_type=jnp.float32)
          + te_ref[...])
    h_out_ref[...] = h0
    a_ref[...] = jnp.dot(h0, ew1a_ref[...],
                         preferred_element_type=jnp.float32) + eb1_ref[...]
    b_out_ref[...] = jnp.dot(h0, ew1b_ref[...],
                             preferred_element_type=jnp.float32)


def _prologue(h, ew, te_row, ew1a, ew1b, eb1):
    grid = (N // BN,)
    nb = lambda i: (i, 0)
    wb = lambda i: (0, 0)
    return pl.pallas_call(
        _pro_body,
        grid=grid,
        in_specs=[
            pl.BlockSpec((BN, HID), nb),
            pl.BlockSpec((HID, HID), wb),
            pl.BlockSpec((1, HID), wb),
            pl.BlockSpec((HID, HID), wb),
            pl.BlockSpec((HID, HID), wb),
            pl.BlockSpec((1, HID), wb),
        ],
        out_specs=[pl.BlockSpec((BN, HID), nb)] * 3,
        out_shape=[jax.ShapeDtypeStruct((N, HID), jnp.float32)] * 3,
    )(h, ew, te_row, ew1a, ew1b, eb1)


def _out_body(h_ref, w1_ref, b1_ref, w2_ref, b2_ref, o_ref):
    z = _silu(jnp.dot(h_ref[...], w1_ref[...],
                      preferred_element_type=jnp.float32) + b1_ref[...])
    o_ref[...] = jnp.dot(z, w2_ref[...],
                         preferred_element_type=jnp.float32) + b2_ref[...]


def _out_mlp(h, w1, b1, w2, b2):
    grid = (N // BN,)
    nb = lambda i: (i, 0)
    wb = lambda i: (0, 0)
    return pl.pallas_call(
        _out_body,
        grid=grid,
        in_specs=[
            pl.BlockSpec((BN, HID), nb),
            pl.BlockSpec((HID, HID), wb),
            pl.BlockSpec((1, HID), wb),
            pl.BlockSpec((HID, HID), wb),
            pl.BlockSpec((1, HID), wb),
        ],
        out_specs=pl.BlockSpec((BN, HID), nb),
        out_shape=jax.ShapeDtypeStruct((N, HID), jnp.float32),
    )(h, w1, b1, w2, b2)


def _sinusoidal(t, dim):
    half = dim // 2
    emb = math.log(10000.0) / (half - 1)
    emb = jnp.exp(jnp.arange(half, dtype=jnp.float32) * -emb)
    e = t[:, None] * emb[None, :]
    return jnp.concatenate([jnp.sin(e), jnp.cos(e)], axis=-1)


def kernel(h, x, params, edge_index, edge_attr, t):
    p = params
    row, col = edge_index[0], edge_index[1]

    te = _sinusoidal(t, TDIM)
    te = _silu(te @ p['time_w1'] + p['time_b1'])
    te = te @ p['time_w2'] + p['time_b2']
    te_row = te[0:1]

    layers = p['layers']
    lp0 = layers[0]
    h_cur, A, B = _prologue(h, p['embed_w'],
                            te_row + p['embed_b'][None, :],
                            lp0['edge_w1'][:HID],
                            lp0['edge_w1'][HID:2 * HID],
                            lp0['edge_b1'][None, :])
    x_cur = x
    for li, lp in enumerate(layers):
        ein_pre = A[row] + B[col]
        xdiff = x_cur[row] - x_cur[col]
        w1t = lp['edge_w1'][2 * HID:]
        m, cdw = _edge_mlp(ein_pre, xdiff, edge_attr, w1t,
                           lp['edge_w2'], lp['edge_b2'][None, :],
                           lp['att_w'], lp['att_b'][None, :],
                           lp['coord_w1'], lp['coord_b1'][None, :],
                           lp['coord_w2'])
        m_i = jnp.zeros((N, HID), jnp.float32).at[row].add(m)
        x_cur = x_cur.at[row].add(cdw)
        if li + 1 < len(layers):
            lpn = layers[li + 1]
            ew1a = lpn['edge_w1'][:HID]
            ew1b = lpn['edge_w1'][HID:2 * HID]
            eb1 = lpn['edge_b1'][None, :]
        else:
            ew1a = lp['edge_w1'][:HID]
            ew1b = lp['edge_w1'][HID:2 * HID]
            eb1 = lp['edge_b1'][None, :]
        h_cur, A, B = _node_update(
            h_cur, m_i,
            lp['node_w1'][:HID], lp['node_w1'][HID:], lp['node_b1'][None, :],
            lp['node_w2'], lp['node_b2'][None, :],
            lp['ln_g'][None, :], lp['ln_b'][None, :],
            ew1a, ew1b, eb1)

    h_out = _out_mlp(h_cur, p['out_w1'], p['out_b1'][None, :],
                     p['out_w2'], p['out_b2'][None, :])
    return h_out, x_cur - x


# speedup vs baseline: 2.0060x; 1.8004x over previous
import functools
import math

import jax
import jax.numpy as jnp
from jax import lax
from jax.experimental import pallas as pl
from jax.experimental.pallas import tpu as pltpu
from jax.experimental.pallas import tpu_sc as plsc

N = 10000
E = 320000
HID = 128
EDIM = 16
TDIM = 64

NP = 10112
EP = 327680
NIDX = 2560
KW = 80
NPH = 5056
NPA = 5120
RT2 = 320
KT = 160

BE = 2048
BN = 1000


def _silu(v):
    return v * jax.nn.sigmoid(v)


def _sc_gather_body(a_hbm, b_hbm, x4_hbm, ridx_hbm, cidx_hbm,
                    g1_hbm, g2_hbm, xd_hbm,
                    ridx_v, cidx_v, xtab, gbufa, gbufb, xdbuf, sema, semb):
    c = lax.axis_index("c")
    s = lax.axis_index("s")
    w = s * 2 + c
    base = w * KW
    pltpu.sync_copy(ridx_hbm.at[pl.ds(base, KW)], ridx_v)
    pltpu.sync_copy(cidx_hbm.at[pl.ds(base, KW)], cidx_v)
    pltpu.sync_copy(x4_hbm, xtab)
    lanes = lax.iota(jnp.int32, 16)

    def body(j, carry):
        cpa = pltpu.async_copy(a_hbm.at[ridx_v.at[j]], gbufa, sema)
        cpb = pltpu.async_copy(b_hbm.at[cidx_v.at[j]], gbufb, semb)
        rrow = ridx_v.at[j]
        crow = cidx_v.at[j]
        for g in range(8):
            r16 = rrow[pl.ds(g * 16, 16)] * 4
            c16 = crow[pl.ds(g * 16, 16)] * 4
            pos = (g * 16 + lanes) * 4
            rad = None
            for k in range(3):
                xr = plsc.load_gather(xtab, [r16 + k])
                xc = plsc.load_gather(xtab, [c16 + k])
                d = xr - xc
                plsc.store_scatter(xdbuf, [pos + k], d)
                rad = d * d if rad is None else rad + d * d
            plsc.store_scatter(xdbuf, [pos + 3], rad)
        cpa.wait()
        cpb.wait()
        ebase = (base + j) * 128
        pltpu.sync_copy(gbufa, g1_hbm.at[pl.ds(ebase, 128)])
        pltpu.sync_copy(gbufb, g2_hbm.at[pl.ds(ebase, 128)])
        pltpu.sync_copy(xdbuf, xd_hbm.at[pl.ds(ebase * 4, 512)])
        return carry

    lax.fori_loop(0, KW, body, 0)


def _sc_gather(a, b, x4, ridx, cidx):
    mesh = plsc.VectorSubcoreMesh(core_axis_name="c", subcore_axis_name="s")
    f = pl.kernel(
        _sc_gather_body,
        out_type=[
            jax.ShapeDtypeStruct((EP, 128), jnp.float32),
            jax.ShapeDtypeStruct((EP, 128), jnp.float32),
            jax.ShapeDtypeStruct((EP * 4,), jnp.float32),
        ],
        mesh=mesh,
        scratch_types=[
            pltpu.VMEM((KW, 128), jnp.int32),
            pltpu.VMEM((KW, 128), jnp.int32),
            pltpu.VMEM((NP * 4,), jnp.float32),
            pltpu.VMEM((128, 128), jnp.float32),
            pltpu.VMEM((128, 128), jnp.float32),
            pltpu.VMEM((512,), jnp.float32),
            pltpu.SemaphoreType.DMA,
            pltpu.SemaphoreType.DMA,
        ],
        compiler_params=pltpu.CompilerParams(needs_layout_passes=False),
    )
    return f(a, b, x4, ridx, cidx)


def _sc_scatter_body(s_hbm, cd_hbm, ridx_hbm, z_hbm, out_m_hbm, out_x_hbm,
                     ridx_v, sbuf, sxbuf, cdbuf, idxbuf, accm, accx):
    c = lax.axis_index("c")
    s = lax.axis_index("s")
    pltpu.sync_copy(z_hbm.at[pl.ds(s * RT2, RT2)],
                    accm.at[pl.ds(s * RT2, RT2)])
    pltpu.sync_copy(z_hbm.at[pl.ds(s * RT2, RT2)],
                    accx.at[pl.ds(s * RT2, RT2)])
    pltpu.sync_copy(z_hbm.at[pl.ds(0, 128)], sxbuf)
    plsc.subcore_barrier()
    lanes = lax.iota(jnp.int32, 16)
    base_node = c * NPH

    def inner(ji, jo):
        j = jo * 16 + ji
        ebase = (s * KT + j) * 128
        pltpu.sync_copy(s_hbm.at[pl.ds(ebase, 128)], sbuf)
        pltpu.sync_copy(cd_hbm.at[pl.ds(ebase * 4, 512)], cdbuf)
        rrow = ridx_v.at[ji]
        for g in range(8):
            r16 = rrow[pl.ds(g * 16, 16)]
            loc = r16 - base_node
            ok = (loc >= 0) & (loc < NPH)
            idxbuf[pl.ds(g * 16, 16)] = jnp.where(ok, loc, NPH)
            pos = g * 16 + lanes
            for k in range(3):
                v = plsc.load_gather(cdbuf, [pos * 4 + k])
                plsc.store_scatter(sxbuf, [pos, jnp.full((16,), k, jnp.int32)],
                                   v)
        pltpu.sync_copy(sbuf, accm.at[idxbuf], add=True)
        pltpu.sync_copy(sxbuf, accx.at[idxbuf], add=True)
        return jo

    def outer(jo, carry):
        pltpu.sync_copy(ridx_hbm.at[pl.ds(s * KT + jo * 16, 16)], ridx_v)
        lax.fori_loop(0, 16, inner, jo)
        return carry

    lax.fori_loop(0, KT // 16, outer, 0)
    plsc.subcore_barrier()
    pltpu.sync_copy(accm.at[pl.ds(s * RT2, RT2)],
                    out_m_hbm.at[c, pl.ds(s * RT2, RT2)])
    pltpu.sync_copy(accx.at[pl.ds(s * RT2, RT2)],
                    out_x_hbm.at[c, pl.ds(s * RT2, RT2)])


def _sc_scatter(srec, cd, ridx, zeros):
    mesh = plsc.VectorSubcoreMesh(core_axis_name="c", subcore_axis_name="s")
    f = pl.kernel(
        _sc_scatter_body,
        out_type=[
            jax.ShapeDtypeStruct((2, NPA, 128), jnp.float32),
            jax.ShapeDtypeStruct((2, NPA, 128), jnp.float32),
        ],
        mesh=mesh,
        scratch_types=[
            pltpu.VMEM((16, 128), jnp.int32),
            pltpu.VMEM((128, 128), jnp.float32),
            pltpu.VMEM((128, 128), jnp.float32),
            pltpu.VMEM((512,), jnp.float32),
            pltpu.VMEM((128,), jnp.int32),
            pltpu.VMEM_SHARED((NPA, 128), jnp.float32),
            pltpu.VMEM_SHARED((NPA, 128), jnp.float32),
        ],
        compiler_params=pltpu.CompilerParams(needs_layout_passes=False),
    )
    return f(srec, cd, ridx, zeros)


def _edge_body(g1_ref, g2_ref, xd_ref, ea_ref, w1t_ref, w2_ref, b2_ref,
               attw_ref, attb_ref, cw1_ref, cb1_ref, cw2_ref,
               sm_ref, cd_ref):
    ein = g1_ref[...] + g2_ref[...]
    xdb = xd_ref[...]
    radial = xdb[:, 3:4]
    ra = jnp.concatenate([radial, ea_ref[...]], axis=1)
    t1 = ein + jnp.dot(ra, w1t_ref[...],
                       preferred_element_type=jnp.float32)
    m1 = _silu(t1)
    m2 = _silu(jnp.dot(m1, w2_ref[...],
                       preferred_element_type=jnp.float32) + b2_ref[...])
    att = jax.nn.sigmoid(jnp.dot(m2, attw_ref[...],
                                 preferred_element_type=jnp.float32)
                         + attb_ref[0, 0])
    m = m2 * att
    c1 = _silu(jnp.dot(m, cw1_ref[...],
                       preferred_element_type=jnp.float32) + cb1_ref[...])
    cw = jnp.dot(c1, cw2_ref[...], preferred_element_type=jnp.float32)
    sm_ref[...] = m
    cd_ref[...] = xdb * (cw / jnp.sqrt(radial + 1e-8))


def _edge_mlp(g1, g2, xd, eattr, w1t, w2, b2, attw, attb, cw1, cb1, cw2):
    grid = (EP // BE,)
    eb = lambda i: (i, 0)
    wb = lambda i: (0, 0)
    return pl.pallas_call(
        _edge_body,
        grid=grid,
        in_specs=[
            pl.BlockSpec((BE, HID), eb),
            pl.BlockSpec((BE, HID), eb),
            pl.BlockSpec((BE, 4), eb),
            pl.BlockSpec((BE, EDIM), eb),
            pl.BlockSpec((1 + EDIM, HID), wb),
            pl.BlockSpec((HID, HID), wb),
            pl.BlockSpec((1, HID), wb),
            pl.BlockSpec((HID, 1), wb),
            pl.BlockSpec((1, 1), wb),
            pl.BlockSpec((HID, HID), wb),
            pl.BlockSpec((1, HID), wb),
            pl.BlockSpec((HID, 1), wb),
        ],
        out_specs=[
            pl.BlockSpec((BE, HID), eb),
            pl.BlockSpec((BE, 4), eb),
        ],
        out_shape=[
            jax.ShapeDtypeStruct((EP, HID), jnp.float32),
            jax.ShapeDtypeStruct((EP, 4), jnp.float32),
        ],
    )(g1, g2, xd, eattr, w1t, w2, b2, attw, attb, cw1, cb1, cw2)


def _node_body(h_ref, mi_ref, xacc_ref, x4_ref, w1h_ref, w1m_ref, nb1_ref,
               w2_ref, nb2_ref, g_ref, b_ref, ew1a_ref, ew1b_ref, eb1_ref,
               h_out_ref, a_ref, b_out_ref, x4_out_ref):
    mi = mi_ref[...]
    dx3 = xacc_ref[:, :3]
    h = h_ref[...]
    z = (jnp.dot(h, w1h_ref[...], preferred_element_type=jnp.float32)
         + jnp.dot(mi, w1m_ref[...],
                   preferred_element_type=jnp.float32) + nb1_ref[...])
    dh = jnp.dot(_silu(z), w2_ref[...],
                 preferred_element_type=jnp.float32) + nb2_ref[...]
    hn = h + dh
    mu = jnp.mean(hn, axis=1, keepdims=True)
    var = jnp.mean((hn - mu) ** 2, axis=1, keepdims=True)
    hln = (hn - mu) / jnp.sqrt(var + 1e-5) * g_ref[...] + b_ref[...]
    h_out_ref[...] = hln
    a_ref[...] = jnp.dot(hln, ew1a_ref[...],
                         preferred_element_type=jnp.float32) + eb1_ref[...]
    b_out_ref[...] = jnp.dot(hln, ew1b_ref[...],
                             preferred_element_type=jnp.float32)
    zero1 = jnp.zeros((dx3.shape[0], 1), jnp.float32)
    x4_out_ref[...] = x4_ref[...] + jnp.concatenate([dx3, zero1], axis=1)


def _node_update(h, mi, xacc, x4, w1h, w1m, nb1, w2, nb2, g, b,
                 ew1a, ew1b, eb1):
    grid = (N // BN,)
    nb = lambda i: (i, 0)
    wb = lambda i: (0, 0)
    return pl.pallas_call(
        _node_body,
        grid=grid,
        in_specs=[
            pl.BlockSpec((BN, HID), nb),
            pl.BlockSpec((BN, HID), nb),
            pl.BlockSpec((BN, HID), nb),
            pl.BlockSpec((BN, 4), nb),
            pl.BlockSpec((HID, HID), wb),
            pl.BlockSpec((HID, HID), wb),
            pl.BlockSpec((1, HID), wb),
            pl.BlockSpec((HID, HID), wb),
            pl.BlockSpec((1, HID), wb),
            pl.BlockSpec((1, HID), wb),
            pl.BlockSpec((1, HID), wb),
            pl.BlockSpec((HID, HID), wb),
            pl.BlockSpec((HID, HID), wb),
            pl.BlockSpec((1, HID), wb),
        ],
        out_specs=[
            pl.BlockSpec((BN, HID), nb),
            pl.BlockSpec((BN, HID), nb),
            pl.BlockSpec((BN, HID), nb),
            pl.BlockSpec((BN, 4), nb),
        ],
        out_shape=[
            jax.ShapeDtypeStruct((N, HID), jnp.float32),
            jax.ShapeDtypeStruct((N, HID), jnp.float32),
            jax.ShapeDtypeStruct((N, HID), jnp.float32),
            jax.ShapeDtypeStruct((N, 4), jnp.float32),
        ],
    )(h, mi, xacc, x4, w1h, w1m, nb1, w2, nb2, g, b, ew1a, ew1b, eb1)


def _pro_body(h_ref, ew_ref, te_ref, ew1a_ref, ew1b_ref, eb1_ref,
              h_out_ref, a_ref, b_out_ref):
    h0 = (jnp.dot(h_ref[...], ew_ref[...], preferred_element_type=jnp.float32)
          + te_ref[...])
    h_out_ref[...] = h0
    a_ref[...] = jnp.dot(h0, ew1a_ref[...],
                         preferred_element_type=jnp.float32) + eb1_ref[...]
    b_out_ref[...] = jnp.dot(h0, ew1b_ref[...],
                             preferred_element_type=jnp.float32)


def _prologue(h, ew, te_row, ew1a, ew1b, eb1):
    grid = (N // BN,)
    nb = lambda i: (i, 0)
    wb = lambda i: (0, 0)
    return pl.pallas_call(
        _pro_body,
        grid=grid,
        in_specs=[
            pl.BlockSpec((BN, HID), nb),
            pl.BlockSpec((HID, HID), wb),
            pl.BlockSpec((1, HID), wb),
            pl.BlockSpec((HID, HID), wb),
            pl.BlockSpec((HID, HID), wb),
            pl.BlockSpec((1, HID), wb),
        ],
        out_specs=[pl.BlockSpec((BN, HID), nb)] * 3,
        out_shape=[jax.ShapeDtypeStruct((N, HID), jnp.float32)] * 3,
    )(h, ew, te_row, ew1a, ew1b, eb1)


def _out_body(h_ref, w1_ref, b1_ref, w2_ref, b2_ref, o_ref):
    z = _silu(jnp.dot(h_ref[...], w1_ref[...],
                      preferred_element_type=jnp.float32) + b1_ref[...])
    o_ref[...] = jnp.dot(z, w2_ref[...],
                         preferred_element_type=jnp.float32) + b2_ref[...]


def _out_mlp(h, w1, b1, w2, b2):
    grid = (N // BN,)
    nb = lambda i: (i, 0)
    wb = lambda i: (0, 0)
    return pl.pallas_call(
        _out_body,
        grid=grid,
        in_specs=[
            pl.BlockSpec((BN, HID), nb),
            pl.BlockSpec((HID, HID), wb),
            pl.BlockSpec((1, HID), wb),
            pl.BlockSpec((HID, HID), wb),
            pl.BlockSpec((1, HID), wb),
        ],
        out_specs=pl.BlockSpec((BN, HID), nb),
        out_shape=jax.ShapeDtypeStruct((N, HID), jnp.float32),
    )(h, w1, b1, w2, b2)


def _sinusoidal(t, dim):
    half = dim // 2
    emb = math.log(10000.0) / (half - 1)
    emb = jnp.exp(jnp.arange(half, dtype=jnp.float32) * -emb)
    e = t[:, None] * emb[None, :]
    return jnp.concatenate([jnp.sin(e), jnp.cos(e)], axis=-1)


def _pad_rows(a, rows):
    return jnp.pad(a, ((0, rows - a.shape[0]), (0, 0)))


def kernel(h, x, params, edge_index, edge_attr, t):
    p = params
    row, col = edge_index[0], edge_index[1]

    te = _sinusoidal(t, TDIM)
    te = _silu(te @ p['time_w1'] + p['time_b1'])
    te = te @ p['time_w2'] + p['time_b2']
    te_row = te[0:1]

    padv = jnp.full((EP - E,), N, jnp.int32)
    ridx = jnp.concatenate([row, padv]).reshape(NIDX, 128)
    cidx = jnp.concatenate([col, padv]).reshape(NIDX, 128)
    ea_pad = jnp.pad(edge_attr, ((0, EP - E), (0, 0)))
    zeros_acc = jnp.zeros((NPA, HID), jnp.float32)

    layers = p['layers']
    lp0 = layers[0]
    h_cur, A, B = _prologue(h, p['embed_w'],
                            te_row + p['embed_b'][None, :],
                            lp0['edge_w1'][:HID],
                            lp0['edge_w1'][HID:2 * HID],
                            lp0['edge_b1'][None, :])
    x4 = jnp.pad(x, ((0, 0), (0, 1)))
    for li, lp in enumerate(layers):
        g1, g2, xd = _sc_gather(_pad_rows(A, NP), _pad_rows(B, NP),
                                _pad_rows(x4, NP).reshape(-1), ridx, cidx)
        xd = xd.reshape(EP, 4)
        w1t = lp['edge_w1'][2 * HID:]
        srec, cdrec = _edge_mlp(g1, g2, xd, ea_pad, w1t,
                                lp['edge_w2'], lp['edge_b2'][None, :],
                                lp['att_w'], lp['att_b'][None, :],
                                lp['coord_w1'], lp['coord_b1'][None, :],
                                lp['coord_w2'])
        accm, accx = _sc_scatter(srec, cdrec.reshape(-1), ridx, zeros_acc)
        mi_full = jnp.concatenate([accm[0, :NPH], accm[1, :NPH]])
        xacc_full = jnp.concatenate([accx[0, :NPH], accx[1, :NPH]])
        if li + 1 < len(layers):
            lpn = layers[li + 1]
            ew1a = lpn['edge_w1'][:HID]
            ew1b = lpn['edge_w1'][HID:2 * HID]
            eb1 = lpn['edge_b1'][None, :]
        else:
            ew1a = lp['edge_w1'][:HID]
            ew1b = lp['edge_w1'][HID:2 * HID]
            eb1 = lp['edge_b1'][None, :]
        h_cur, A, B, x4 = _node_update(
            h_cur, mi_full, xacc_full, x4,
            lp['node_w1'][:HID], lp['node_w1'][HID:], lp['node_b1'][None, :],
            lp['node_w2'], lp['node_b2'][None, :],
            lp['ln_g'][None, :], lp['ln_b'][None, :],
            ew1a, ew1b, eb1)

    h_out = _out_mlp(h_cur, p['out_w1'], p['out_b1'][None, :],
                     p['out_w2'], p['out_b2'][None, :])
    return h_out, x4[:, :3] - x


# speedup vs baseline: 2.1090x; 1.0513x over previous
import functools
import math

import jax
import jax.numpy as jnp
from jax import lax
from jax.experimental import pallas as pl
from jax.experimental.pallas import tpu as pltpu
from jax.experimental.pallas import tpu_sc as plsc

N = 10000
E = 320000
HID = 128
EDIM = 16
TDIM = 64

NP = 10112
EP = 327680
NIDX = 2560
KW = 80
NPH = 5056
NPA = 5120
RT2 = 320
KT = 160

BE = 2048
BN = 1000


def _silu(v):
    return v * jax.nn.sigmoid(v)


def _sc_gather_body(a_hbm, b_hbm, x4_hbm, ridx_hbm, cidx_hbm,
                    g1_hbm, g2_hbm, xd_hbm,
                    ridx_v, cidx_v, xtab,
                    gbufa0, gbufb0, xdbuf0, gbufa1, gbufb1, xdbuf1,
                    sa0, sb0, sa1, sb1, swa0, swb0, swx0, swa1, swb1, swx1):
    c = lax.axis_index("c")
    s = lax.axis_index("s")
    w = s * 2 + c
    base = w * KW
    pltpu.sync_copy(x4_hbm, xtab)
    lanes = lax.iota(jnp.int32, 16)

    def xd_compute(rrow, crow, xdbuf):
        for g in range(8):
            r16 = rrow[pl.ds(g * 16, 16)] * 4
            c16 = crow[pl.ds(g * 16, 16)] * 4
            pos = (g * 16 + lanes) * 4
            rad = None
            for k in range(3):
                xr = plsc.load_gather(xtab, [r16 + k])
                xc = plsc.load_gather(xtab, [c16 + k])
                d = xr - xc
                plsc.store_scatter(xdbuf, [pos + k], d)
                rad = d * d if rad is None else rad + d * d
            plsc.store_scatter(xdbuf, [pos + 3], rad)

    def pair(jo, carry):
        jb = jo * 2
        cpa0 = pltpu.async_copy(a_hbm.at[ridx_v.at[jb]], gbufa0, sa0)
        cpb0 = pltpu.async_copy(b_hbm.at[cidx_v.at[jb]], gbufb0, sb0)
        cpa1 = pltpu.async_copy(a_hbm.at[ridx_v.at[jb + 1]], gbufa1, sa1)
        cpb1 = pltpu.async_copy(b_hbm.at[cidx_v.at[jb + 1]], gbufb1, sb1)
        xd_compute(ridx_v.at[jb], cidx_v.at[jb], xdbuf0)
        xd_compute(ridx_v.at[jb + 1], cidx_v.at[jb + 1], xdbuf1)
        ebase0 = (carry + jb) * 128
        ebase1 = (carry + jb + 1) * 128
        cpa0.wait()
        cpb0.wait()
        w10 = pltpu.async_copy(gbufa0, g1_hbm.at[pl.ds(ebase0, 128)], swa0)
        w20 = pltpu.async_copy(gbufb0, g2_hbm.at[pl.ds(ebase0, 128)], swb0)
        w30 = pltpu.async_copy(xdbuf0, xd_hbm.at[pl.ds(ebase0 * 4, 512)], swx0)
        cpa1.wait()
        cpb1.wait()
        w11 = pltpu.async_copy(gbufa1, g1_hbm.at[pl.ds(ebase1, 128)], swa1)
        w21 = pltpu.async_copy(gbufb1, g2_hbm.at[pl.ds(ebase1, 128)], swb1)
        w31 = pltpu.async_copy(xdbuf1, xd_hbm.at[pl.ds(ebase1 * 4, 512)], swx1)
        w10.wait()
        w20.wait()
        w30.wait()
        w11.wait()
        w21.wait()
        w31.wait()
        return carry

    def block(bo, carry):
        pltpu.sync_copy(ridx_hbm.at[pl.ds(base + bo * 16, 16)], ridx_v)
        pltpu.sync_copy(cidx_hbm.at[pl.ds(base + bo * 16, 16)], cidx_v)
        lax.fori_loop(0, 8, pair, base + bo * 16)
        return carry

    lax.fori_loop(0, KW // 16, block, 0)


def _sc_gather(a, b, x4, ridx, cidx):
    mesh = plsc.VectorSubcoreMesh(core_axis_name="c", subcore_axis_name="s")
    f = pl.kernel(
        _sc_gather_body,
        out_type=[
            jax.ShapeDtypeStruct((EP, 128), jnp.float32),
            jax.ShapeDtypeStruct((EP, 128), jnp.float32),
            jax.ShapeDtypeStruct((EP * 4,), jnp.float32),
        ],
        mesh=mesh,
        scratch_types=[
            pltpu.VMEM((16, 128), jnp.int32),
            pltpu.VMEM((16, 128), jnp.int32),
            pltpu.VMEM((NP * 4,), jnp.float32),
            pltpu.VMEM((128, 128), jnp.float32),
            pltpu.VMEM((128, 128), jnp.float32),
            pltpu.VMEM((512,), jnp.float32),
            pltpu.VMEM((128, 128), jnp.float32),
            pltpu.VMEM((128, 128), jnp.float32),
            pltpu.VMEM((512,), jnp.float32),
        ] + [pltpu.SemaphoreType.DMA] * 10,
        compiler_params=pltpu.CompilerParams(needs_layout_passes=False),
    )
    return f(a, b, x4, ridx, cidx)


def _sc_scatter_body(s_hbm, cd_hbm, ridx_hbm, z_hbm, out_m_hbm, out_x_hbm,
                     ridx_v, sbuf0, sxbuf0, cdbuf0, idxbuf0,
                     sbuf1, sxbuf1, cdbuf1, idxbuf1, accm, accx,
                     sl0, sc0, sl1, sc1, sam0, sax0, sam1, sax1):
    c = lax.axis_index("c")
    s = lax.axis_index("s")
    pltpu.sync_copy(z_hbm.at[pl.ds(s * RT2, RT2)],
                    accm.at[pl.ds(s * RT2, RT2)])
    pltpu.sync_copy(z_hbm.at[pl.ds(s * RT2, RT2)],
                    accx.at[pl.ds(s * RT2, RT2)])
    pltpu.sync_copy(z_hbm.at[pl.ds(0, 64)], sxbuf0)
    pltpu.sync_copy(z_hbm.at[pl.ds(0, 64)], sxbuf1)
    plsc.subcore_barrier()
    lanes = lax.iota(jnp.int32, 16)
    base_node = c * NPH

    def route(rrow, half, idxbuf):
        for g in range(4):
            r16 = rrow[pl.ds(half * 64 + g * 16, 16)]
            loc = r16 - base_node
            ok = (loc >= 0) & (loc < NPH)
            idxbuf[pl.ds(g * 16, 16)] = jnp.where(ok, loc, NPH)

    def expand(cdbuf, sxbuf):
        for g in range(4):
            pos = g * 16 + lanes
            for k in range(3):
                v = plsc.load_gather(cdbuf, [pos * 4 + k])
                plsc.store_scatter(sxbuf, [pos, jnp.full((16,), k, jnp.int32)],
                                   v)

    def inner(ji, jo):
        j = jo * 16 + ji
        ebase = (s * KT + j) * 128
        l0 = pltpu.async_copy(s_hbm.at[pl.ds(ebase, 64)], sbuf0, sl0)
        c0 = pltpu.async_copy(cd_hbm.at[pl.ds(ebase * 4, 256)], cdbuf0, sc0)
        l1 = pltpu.async_copy(s_hbm.at[pl.ds(ebase + 64, 64)], sbuf1, sl1)
        c1 = pltpu.async_copy(cd_hbm.at[pl.ds(ebase * 4 + 256, 256)],
                              cdbuf1, sc1)
        rrow = ridx_v.at[ji]
        route(rrow, 0, idxbuf0)
        route(rrow, 1, idxbuf1)
        l0.wait()
        c0.wait()
        expand(cdbuf0, sxbuf0)
        a0 = pltpu.async_copy(sbuf0, accm.at[idxbuf0], sam0, add=True)
        x0 = pltpu.async_copy(sxbuf0, accx.at[idxbuf0], sax0, add=True)
        l1.wait()
        c1.wait()
        expand(cdbuf1, sxbuf1)
        a1 = pltpu.async_copy(sbuf1, accm.at[idxbuf1], sam1, add=True)
        x1 = pltpu.async_copy(sxbuf1, accx.at[idxbuf1], sax1, add=True)
        a0.wait()
        x0.wait()
        a1.wait()
        x1.wait()
        return jo

    def outer(jo, carry):
        pltpu.sync_copy(ridx_hbm.at[pl.ds(s * KT + jo * 16, 16)], ridx_v)
        lax.fori_loop(0, 16, inner, jo)
        return carry

    lax.fori_loop(0, KT // 16, outer, 0)
    plsc.subcore_barrier()
    pltpu.sync_copy(accm.at[pl.ds(s * RT2, RT2)],
                    out_m_hbm.at[c, pl.ds(s * RT2, RT2)])
    pltpu.sync_copy(accx.at[pl.ds(s * RT2, RT2)],
                    out_x_hbm.at[c, pl.ds(s * RT2, RT2)])


def _sc_scatter(srec, cd, ridx, zeros):
    mesh = plsc.VectorSubcoreMesh(core_axis_name="c", subcore_axis_name="s")
    f = pl.kernel(
        _sc_scatter_body,
        out_type=[
            jax.ShapeDtypeStruct((2, NPA, 128), jnp.float32),
            jax.ShapeDtypeStruct((2, NPA, 128), jnp.float32),
        ],
        mesh=mesh,
        scratch_types=[
            pltpu.VMEM((16, 128), jnp.int32),
            pltpu.VMEM((64, 128), jnp.float32),
            pltpu.VMEM((64, 128), jnp.float32),
            pltpu.VMEM((256,), jnp.float32),
            pltpu.VMEM((64,), jnp.int32),
            pltpu.VMEM((64, 128), jnp.float32),
            pltpu.VMEM((64, 128), jnp.float32),
            pltpu.VMEM((256,), jnp.float32),
            pltpu.VMEM((64,), jnp.int32),
            pltpu.VMEM_SHARED((NPA, 128), jnp.float32),
            pltpu.VMEM_SHARED((NPA, 128), jnp.float32),
        ] + [pltpu.SemaphoreType.DMA] * 8,
        compiler_params=pltpu.CompilerParams(needs_layout_passes=False),
    )
    return f(srec, cd, ridx, zeros)


def _edge_body(g1_ref, g2_ref, xd_ref, ea_ref, w1t_ref, w2_ref, b2_ref,
               attw_ref, attb_ref, cw1_ref, cb1_ref, cw2_ref,
               sm_ref, cd_ref):
    ein = g1_ref[...] + g2_ref[...]
    xdb = xd_ref[...]
    radial = xdb[:, 3:4]
    ra = jnp.concatenate([radial, ea_ref[...]], axis=1)
    t1 = ein + jnp.dot(ra, w1t_ref[...],
                       preferred_element_type=jnp.float32)
    m1 = _silu(t1)
    m2 = _silu(jnp.dot(m1, w2_ref[...],
                       preferred_element_type=jnp.float32) + b2_ref[...])
    att = jax.nn.sigmoid(jnp.dot(m2, attw_ref[...],
                                 preferred_element_type=jnp.float32)
                         + attb_ref[0, 0])
    m = m2 * att
    c1 = _silu(jnp.dot(m, cw1_ref[...],
                       preferred_element_type=jnp.float32) + cb1_ref[...])
    cw = jnp.dot(c1, cw2_ref[...], preferred_element_type=jnp.float32)
    sm_ref[...] = m
    cd_ref[...] = xdb * (cw / jnp.sqrt(radial + 1e-8))


def _edge_mlp(g1, g2, xd, eattr, w1t, w2, b2, attw, attb, cw1, cb1, cw2):
    grid = (EP // BE,)
    eb = lambda i: (i, 0)
    wb = lambda i: (0, 0)
    return pl.pallas_call(
        _edge_body,
        grid=grid,
        in_specs=[
            pl.BlockSpec((BE, HID), eb),
            pl.BlockSpec((BE, HID), eb),
            pl.BlockSpec((BE, 4), eb),
            pl.BlockSpec((BE, EDIM), eb),
            pl.BlockSpec((1 + EDIM, HID), wb),
            pl.BlockSpec((HID, HID), wb),
            pl.BlockSpec((1, HID), wb),
            pl.BlockSpec((HID, 1), wb),
            pl.BlockSpec((1, 1), wb),
            pl.BlockSpec((HID, HID), wb),
            pl.BlockSpec((1, HID), wb),
            pl.BlockSpec((HID, 1), wb),
        ],
        out_specs=[
            pl.BlockSpec((BE, HID), eb),
            pl.BlockSpec((BE, 4), eb),
        ],
        out_shape=[
            jax.ShapeDtypeStruct((EP, HID), jnp.float32),
            jax.ShapeDtypeStruct((EP, 4), jnp.float32),
        ],
    )(g1, g2, xd, eattr, w1t, w2, b2, attw, attb, cw1, cb1, cw2)


def _node_body(h_ref, mi_ref, xacc_ref, x4_ref, w1h_ref, w1m_ref, nb1_ref,
               w2_ref, nb2_ref, g_ref, b_ref, ew1a_ref, ew1b_ref, eb1_ref,
               h_out_ref, a_ref, b_out_ref, x4_out_ref):
    mi = mi_ref[...]
    dx3 = xacc_ref[:, :3]
    h = h_ref[...]
    z = (jnp.dot(h, w1h_ref[...], preferred_element_type=jnp.float32)
         + jnp.dot(mi, w1m_ref[...],
                   preferred_element_type=jnp.float32) + nb1_ref[...])
    dh = jnp.dot(_silu(z), w2_ref[...],
                 preferred_element_type=jnp.float32) + nb2_ref[...]
    hn = h + dh
    mu = jnp.mean(hn, axis=1, keepdims=True)
    var = jnp.mean((hn - mu) ** 2, axis=1, keepdims=True)
    hln = (hn - mu) / jnp.sqrt(var + 1e-5) * g_ref[...] + b_ref[...]
    h_out_ref[...] = hln
    a_ref[...] = jnp.dot(hln, ew1a_ref[...],
                         preferred_element_type=jnp.float32) + eb1_ref[...]
    b_out_ref[...] = jnp.dot(hln, ew1b_ref[...],
                             preferred_element_type=jnp.float32)
    zero1 = jnp.zeros((dx3.shape[0], 1), jnp.float32)
    x4_out_ref[...] = x4_ref[...] + jnp.concatenate([dx3, zero1], axis=1)


def _node_update(h, mi, xacc, x4, w1h, w1m, nb1, w2, nb2, g, b,
                 ew1a, ew1b, eb1):
    grid = (N // BN,)
    nb = lambda i: (i, 0)
    wb = lambda i: (0, 0)
    return pl.pallas_call(
        _node_body,
        grid=grid,
        in_specs=[
            pl.BlockSpec((BN, HID), nb),
            pl.BlockSpec((BN, HID), nb),
            pl.BlockSpec((BN, HID), nb),
            pl.BlockSpec((BN, 4), nb),
            pl.BlockSpec((HID, HID), wb),
            pl.BlockSpec((HID, HID), wb),
            pl.BlockSpec((1, HID), wb),
            pl.BlockSpec((HID, HID), wb),
            pl.BlockSpec((1, HID), wb),
            pl.BlockSpec((1, HID), wb),
            pl.BlockSpec((1, HID), wb),
            pl.BlockSpec((HID, HID), wb),
            pl.BlockSpec((HID, HID), wb),
            pl.BlockSpec((1, HID), wb),
        ],
        out_specs=[
            pl.BlockSpec((BN, HID), nb),
            pl.BlockSpec((BN, HID), nb),
            pl.BlockSpec((BN, HID), nb),
            pl.BlockSpec((BN, 4), nb),
        ],
        out_shape=[
            jax.ShapeDtypeStruct((N, HID), jnp.float32),
            jax.ShapeDtypeStruct((N, HID), jnp.float32),
            jax.ShapeDtypeStruct((N, HID), jnp.float32),
            jax.ShapeDtypeStruct((N, 4), jnp.float32),
        ],
    )(h, mi, xacc, x4, w1h, w1m, nb1, w2, nb2, g, b, ew1a, ew1b, eb1)


def _pro_body(h_ref, ew_ref, te_ref, ew1a_ref, ew1b_ref, eb1_ref,
              h_out_ref, a_ref, b_out_ref):
    h0 = (jnp.dot(h_ref[...], ew_ref[...], preferred_element_type=jnp.float32)
          + te_ref[...])
    h_out_ref[...] = h0
    a_ref[...] = jnp.dot(h0, ew1a_ref[...],
                         preferred_element_type=jnp.float32) + eb1_ref[...]
    b_out_ref[...] = jnp.dot(h0, ew1b_ref[...],
                             preferred_element_type=jnp.float32)


def _prologue(h, ew, te_row, ew1a, ew1b, eb1):
    grid = (N // BN,)
    nb = lambda i: (i, 0)
    wb = lambda i: (0, 0)
    return pl.pallas_call(
        _pro_body,
        grid=grid,
        in_specs=[
            pl.BlockSpec((BN, HID), nb),
            pl.BlockSpec((HID, HID), wb),
            pl.BlockSpec((1, HID), wb),
            pl.BlockSpec((HID, HID), wb),
            pl.BlockSpec((HID, HID), wb),
            pl.BlockSpec((1, HID), wb),
        ],
        out_specs=[pl.BlockSpec((BN, HID), nb)] * 3,
        out_shape=[jax.ShapeDtypeStruct((N, HID), jnp.float32)] * 3,
    )(h, ew, te_row, ew1a, ew1b, eb1)


def _out_body(h_ref, w1_ref, b1_ref, w2_ref, b2_ref, o_ref):
    z = _silu(jnp.dot(h_ref[...], w1_ref[...],
                      preferred_element_type=jnp.float32) + b1_ref[...])
    o_ref[...] = jnp.dot(z, w2_ref[...],
                         preferred_element_type=jnp.float32) + b2_ref[...]


def _out_mlp(h, w1, b1, w2, b2):
    grid = (N // BN,)
    nb = lambda i: (i, 0)
    wb = lambda i: (0, 0)
    return pl.pallas_call(
        _out_body,
        grid=grid,
        in_specs=[
            pl.BlockSpec((BN, HID), nb),
            pl.BlockSpec((HID, HID), wb),
            pl.BlockSpec((1, HID), wb),
            pl.BlockSpec((HID, HID), wb),
            pl.BlockSpec((1, HID), wb),
        ],
        out_specs=pl.BlockSpec((BN, HID), nb),
        out_shape=jax.ShapeDtypeStruct((N, HID), jnp.float32),
    )(h, w1, b1, w2, b2)


def _sinusoidal(t, dim):
    half = dim // 2
    emb = math.log(10000.0) / (half - 1)
    emb = jnp.exp(jnp.arange(half, dtype=jnp.float32) * -emb)
    e = t[:, None] * emb[None, :]
    return jnp.concatenate([jnp.sin(e), jnp.cos(e)], axis=-1)


def _pad_rows(a, rows):
    return jnp.pad(a, ((0, rows - a.shape[0]), (0, 0)))


def kernel(h, x, params, edge_index, edge_attr, t):
    p = params
    row, col = edge_index[0], edge_index[1]

    te = _sinusoidal(t, TDIM)
    te = _silu(te @ p['time_w1'] + p['time_b1'])
    te = te @ p['time_w2'] + p['time_b2']
    te_row = te[0:1]

    padv = jnp.full((EP - E,), N, jnp.int32)
    ridx = jnp.concatenate([row, padv]).reshape(NIDX, 128)
    cidx = jnp.concatenate([col, padv]).reshape(NIDX, 128)
    ea_pad = jnp.pad(edge_attr, ((0, EP - E), (0, 0)))
    zeros_acc = jnp.zeros((NPA, HID), jnp.float32)

    layers = p['layers']
    lp0 = layers[0]
    h_cur, A, B = _prologue(h, p['embed_w'],
                            te_row + p['embed_b'][None, :],
                            lp0['edge_w1'][:HID],
                            lp0['edge_w1'][HID:2 * HID],
                            lp0['edge_b1'][None, :])
    x4 = jnp.pad(x, ((0, 0), (0, 1)))
    for li, lp in enumerate(layers):
        g1, g2, xd = _sc_gather(_pad_rows(A, NP), _pad_rows(B, NP),
                                _pad_rows(x4, NP).reshape(-1), ridx, cidx)
        xd = xd.reshape(EP, 4)
        w1t = lp['edge_w1'][2 * HID:]
        srec, cdrec = _edge_mlp(g1, g2, xd, ea_pad, w1t,
                                lp['edge_w2'], lp['edge_b2'][None, :],
                                lp['att_w'], lp['att_b'][None, :],
                                lp['coord_w1'], lp['coord_b1'][None, :],
                                lp['coord_w2'])
        accm, accx = _sc_scatter(srec, cdrec.reshape(-1), ridx, zeros_acc)
        mi_full = jnp.concatenate([accm[0, :NPH], accm[1, :NPH]])
        xacc_full = jnp.concatenate([accx[0, :NPH], accx[1, :NPH]])
        if li + 1 < len(layers):
            lpn = layers[li + 1]
            ew1a = lpn['edge_w1'][:HID]
            ew1b = lpn['edge_w1'][HID:2 * HID]
            eb1 = lpn['edge_b1'][None, :]
        else:
            ew1a = lp['edge_w1'][:HID]
            ew1b = lp['edge_w1'][HID:2 * HID]
            eb1 = lp['edge_b1'][None, :]
        h_cur, A, B, x4 = _node_update(
            h_cur, mi_full, xacc_full, x4,
            lp['node_w1'][:HID], lp['node_w1'][HID:], lp['node_b1'][None, :],
            lp['node_w2'], lp['node_b2'][None, :],
            lp['ln_g'][None, :], lp['ln_b'][None, :],
            ew1a, ew1b, eb1)

    h_out = _out_mlp(h_cur, p['out_w1'], p['out_b1'][None, :],
                     p['out_w2'], p['out_b2'][None, :])
    return h_out, x4[:, :3] - x


# speedup vs baseline: 2.1269x; 1.0085x over previous
import math

import jax
import jax.numpy as jnp
from jax import lax
from jax.experimental import pallas as pl
from jax.experimental.pallas import tpu as pltpu
from jax.experimental.pallas import tpu_sc as plsc

N = 10000
E = 320000
HID = 128
EDIM = 16
TDIM = 64

NP = 10112
EP = 327680
NIDX = 2560
KW = 80
NPH = 5000
NPA = 5120
RT2 = 320
KT = 160

BE = 2048
BN = 1000


def _silu(v):
    return v * jax.nn.sigmoid(v)


def _sc_gather_body(a_hbm, b_hbm, x4_hbm, ridx_hbm, cidx_hbm,
                    g1_hbm, g2_hbm, xd_hbm,
                    ridx_v, cidx_v, xtab,
                    gbufa0, gbufb0, xdbuf0, gbufa1, gbufb1, xdbuf1,
                    sa0, sb0, sa1, sb1, swa0, swb0, swx0, swa1, swb1, swx1):
    c = lax.axis_index("c")
    s = lax.axis_index("s")
    w = s * 2 + c
    base = w * KW
    pltpu.sync_copy(x4_hbm, xtab)
    lanes = lax.iota(jnp.int32, 16)

    def xd_compute(rrow, crow, xdbuf):
        for g in range(8):
            r16 = rrow[pl.ds(g * 16, 16)] * 4
            c16 = crow[pl.ds(g * 16, 16)] * 4
            pos = (g * 16 + lanes) * 4
            rad = None
            for k in range(3):
                xr = plsc.load_gather(xtab, [r16 + k])
                xc = plsc.load_gather(xtab, [c16 + k])
                d = xr - xc
                plsc.store_scatter(xdbuf, [pos + k], d)
                rad = d * d if rad is None else rad + d * d
            plsc.store_scatter(xdbuf, [pos + 3], rad)

    def pair(jo, blkbase):
        jb = jo * 2
        cpa0 = pltpu.async_copy(a_hbm.at[ridx_v.at[jb]], gbufa0, sa0)
        cpb0 = pltpu.async_copy(b_hbm.at[cidx_v.at[jb]], gbufb0, sb0)
        cpa1 = pltpu.async_copy(a_hbm.at[ridx_v.at[jb + 1]], gbufa1, sa1)
        cpb1 = pltpu.async_copy(b_hbm.at[cidx_v.at[jb + 1]], gbufb1, sb1)
        xd_compute(ridx_v.at[jb], cidx_v.at[jb], xdbuf0)
        xd_compute(ridx_v.at[jb + 1], cidx_v.at[jb + 1], xdbuf1)
        ebase0 = (blkbase + jb) * 128
        ebase1 = (blkbase + jb + 1) * 128
        cpa0.wait()
        cpb0.wait()
        w10 = pltpu.async_copy(gbufa0, g1_hbm.at[pl.ds(ebase0, 128)], swa0)
        w20 = pltpu.async_copy(gbufb0, g2_hbm.at[pl.ds(ebase0, 128)], swb0)
        w30 = pltpu.async_copy(xdbuf0, xd_hbm.at[pl.ds(ebase0 * 4, 512)], swx0)
        cpa1.wait()
        cpb1.wait()
        w11 = pltpu.async_copy(gbufa1, g1_hbm.at[pl.ds(ebase1, 128)], swa1)
        w21 = pltpu.async_copy(gbufb1, g2_hbm.at[pl.ds(ebase1, 128)], swb1)
        w31 = pltpu.async_copy(xdbuf1, xd_hbm.at[pl.ds(ebase1 * 4, 512)], swx1)
        w10.wait()
        w20.wait()
        w30.wait()
        w11.wait()
        w21.wait()
        w31.wait()
        return blkbase

    def block(bo, carry):
        pltpu.sync_copy(ridx_hbm.at[pl.ds(base + bo * 16, 16)], ridx_v)
        pltpu.sync_copy(cidx_hbm.at[pl.ds(base + bo * 16, 16)], cidx_v)
        lax.fori_loop(0, 8, pair, base + bo * 16)
        return carry

    lax.fori_loop(0, KW // 16, block, 0)


def _sc_gather(a, b, x4, ridx, cidx):
    mesh = plsc.VectorSubcoreMesh(core_axis_name="c", subcore_axis_name="s")
    f = pl.kernel(
        _sc_gather_body,
        out_type=[
            jax.ShapeDtypeStruct((EP, 128), jnp.float32),
            jax.ShapeDtypeStruct((EP, 128), jnp.float32),
            jax.ShapeDtypeStruct((EP * 4,), jnp.float32),
        ],
        mesh=mesh,
        scratch_types=[
            pltpu.VMEM((16, 128), jnp.int32),
            pltpu.VMEM((16, 128), jnp.int32),
            pltpu.VMEM((NP * 4,), jnp.float32),
            pltpu.VMEM((128, 128), jnp.float32),
            pltpu.VMEM((128, 128), jnp.float32),
            pltpu.VMEM((512,), jnp.float32),
            pltpu.VMEM((128, 128), jnp.float32),
            pltpu.VMEM((128, 128), jnp.float32),
            pltpu.VMEM((512,), jnp.float32),
        ] + [pltpu.SemaphoreType.DMA] * 10,
        compiler_params=pltpu.CompilerParams(needs_layout_passes=False),
    )
    return f(a, b, x4, ridx, cidx)


def _sc_scatter_body(s_hbm, cd_hbm, ridx_hbm, z_hbm, out_m_hbm, out_x_hbm,
                     ridx_v, sbuf0, sxbuf0, cdbuf0, idxbuf0,
                     sbuf1, sxbuf1, cdbuf1, idxbuf1, accm, accx,
                     sl0, sc0, sl1, sc1, sam0, sax0, sam1, sax1):
    c = lax.axis_index("c")
    s = lax.axis_index("s")
    pltpu.sync_copy(z_hbm.at[pl.ds(s * RT2, RT2)],
                    accm.at[pl.ds(s * RT2, RT2)])
    pltpu.sync_copy(z_hbm.at[pl.ds(s * RT2, RT2)],
                    accx.at[pl.ds(s * RT2, RT2)])
    pltpu.sync_copy(z_hbm.at[pl.ds(0, 64)], sxbuf0)
    pltpu.sync_copy(z_hbm.at[pl.ds(0, 64)], sxbuf1)
    plsc.subcore_barrier()
    lanes = lax.iota(jnp.int32, 16)
    base_node = c * NPH

    def route(rrow, half, idxbuf):
        for g in range(4):
            r16 = rrow[pl.ds(half * 64 + g * 16, 16)]
            loc = r16 - base_node
            ok = (loc >= 0) & (loc < NPH)
            idxbuf[pl.ds(g * 16, 16)] = jnp.where(ok, loc, NPH)

    def expand(cdbuf, sxbuf):
        for g in range(4):
            pos = g * 16 + lanes
            for k in range(3):
                v = plsc.load_gather(cdbuf, [pos * 4 + k])
                plsc.store_scatter(sxbuf, [pos, jnp.full((16,), k, jnp.int32)],
                                   v)

    def inner(ji, jo):
        j = jo * 16 + ji
        ebase = (s * KT + j) * 128
        l0 = pltpu.async_copy(s_hbm.at[pl.ds(ebase, 64)], sbuf0, sl0)
        c0 = pltpu.async_copy(cd_hbm.at[pl.ds(ebase * 4, 256)], cdbuf0, sc0)
        l1 = pltpu.async_copy(s_hbm.at[pl.ds(ebase + 64, 64)], sbuf1, sl1)
        c1 = pltpu.async_copy(cd_hbm.at[pl.ds(ebase * 4 + 256, 256)],
                              cdbuf1, sc1)
        rrow = ridx_v.at[ji]
        route(rrow, 0, idxbuf0)
        route(rrow, 1, idxbuf1)
        l0.wait()
        c0.wait()
        expand(cdbuf0, sxbuf0)
        a0 = pltpu.async_copy(sbuf0, accm.at[idxbuf0], sam0, add=True)
        x0 = pltpu.async_copy(sxbuf0, accx.at[idxbuf0], sax0, add=True)
        l1.wait()
        c1.wait()
        expand(cdbuf1, sxbuf1)
        a1 = pltpu.async_copy(sbuf1, accm.at[idxbuf1], sam1, add=True)
        x1 = pltpu.async_copy(sxbuf1, accx.at[idxbuf1], sax1, add=True)
        a0.wait()
        x0.wait()
        a1.wait()
        x1.wait()
        return jo

    def outer(jo, carry):
        pltpu.sync_copy(ridx_hbm.at[pl.ds(s * KT + jo * 16, 16)], ridx_v)
        lax.fori_loop(0, 16, inner, jo)
        return carry

    lax.fori_loop(0, KT // 16, outer, 0)
    plsc.subcore_barrier()
    pltpu.sync_copy(accm.at[pl.ds(s * RT2, RT2)],
                    out_m_hbm.at[c, pl.ds(s * RT2, RT2)])
    pltpu.sync_copy(accx.at[pl.ds(s * RT2, RT2)],
                    out_x_hbm.at[c, pl.ds(s * RT2, RT2)])


def _sc_scatter(srec, cd, ridx, zeros):
    mesh = plsc.VectorSubcoreMesh(core_axis_name="c", subcore_axis_name="s")
    f = pl.kernel(
        _sc_scatter_body,
        out_type=[
            jax.ShapeDtypeStruct((2, NPA, 128), jnp.float32),
            jax.ShapeDtypeStruct((2, NPA, 128), jnp.float32),
        ],
        mesh=mesh,
        scratch_types=[
            pltpu.VMEM((16, 128), jnp.int32),
            pltpu.VMEM((64, 128), jnp.float32),
            pltpu.VMEM((64, 128), jnp.float32),
            pltpu.VMEM((256,), jnp.float32),
            pltpu.VMEM((64,), jnp.int32),
            pltpu.VMEM((64, 128), jnp.float32),
            pltpu.VMEM((64, 128), jnp.float32),
            pltpu.VMEM((256,), jnp.float32),
            pltpu.VMEM((64,), jnp.int32),
            pltpu.VMEM_SHARED((NPA, 128), jnp.float32),
            pltpu.VMEM_SHARED((NPA, 128), jnp.float32),
        ] + [pltpu.SemaphoreType.DMA] * 8,
        compiler_params=pltpu.CompilerParams(needs_layout_passes=False),
    )
    return f(srec, cd, ridx, zeros)


def _edge_body(g1_ref, g2_ref, xd_ref, ea_ref, w1t_ref, w2_ref, b2_ref,
               attw_ref, attb_ref, cw1_ref, cb1_ref, cw2_ref,
               sm_ref, cd_ref):
    ein = g1_ref[...] + g2_ref[...]
    xdb = xd_ref[...]
    radial = xdb[:, 3:4]
    ra = jnp.concatenate([radial, ea_ref[...]], axis=1)
    t1 = ein + jnp.dot(ra, w1t_ref[...],
                       preferred_element_type=jnp.float32)
    m1 = _silu(t1)
    m2 = _silu(jnp.dot(m1.astype(jnp.bfloat16),
                       w2_ref[...].astype(jnp.bfloat16),
                       preferred_element_type=jnp.float32) + b2_ref[...])
    att = jax.nn.sigmoid(jnp.dot(m2, attw_ref[...],
                                 preferred_element_type=jnp.float32)
                         + attb_ref[0, 0])
    m = m2 * att
    c1 = _silu(jnp.dot(m.astype(jnp.bfloat16),
                       cw1_ref[...].astype(jnp.bfloat16),
                       preferred_element_type=jnp.float32) + cb1_ref[...])
    cw = jnp.dot(c1, cw2_ref[...], preferred_element_type=jnp.float32)
    sm_ref[...] = m
    cd_ref[...] = xdb * (cw / jnp.sqrt(radial + 1e-8))


def _edge_mlp(g1, g2, xd, eattr, w1t, w2, b2, attw, attb, cw1, cb1, cw2):
    grid = (EP // BE,)
    eb = lambda i: (i, 0)
    wb = lambda i: (0, 0)
    return pl.pallas_call(
        _edge_body,
        grid=grid,
        in_specs=[
            pl.BlockSpec((BE, HID), eb),
            pl.BlockSpec((BE, HID), eb),
            pl.BlockSpec((BE, 4), eb),
            pl.BlockSpec((BE, EDIM), eb),
            pl.BlockSpec((1 + EDIM, HID), wb),
            pl.BlockSpec((HID, HID), wb),
            pl.BlockSpec((1, HID), wb),
            pl.BlockSpec((HID, 1), wb),
            pl.BlockSpec((1, 1), wb),
            pl.BlockSpec((HID, HID), wb),
            pl.BlockSpec((1, HID), wb),
            pl.BlockSpec((HID, 1), wb),
        ],
        out_specs=[
            pl.BlockSpec((BE, HID), eb),
            pl.BlockSpec((BE, 4), eb),
        ],
        out_shape=[
            jax.ShapeDtypeStruct((EP, HID), jnp.float32),
            jax.ShapeDtypeStruct((EP, 4), jnp.float32),
        ],
    )(g1, g2, xd, eattr, w1t, w2, b2, attw, attb, cw1, cb1, cw2)


def _node_body(h_ref, mi_ref, xacc_ref, x4_ref, w1h_ref, w1m_ref, nb1_ref,
               w2_ref, nb2_ref, g_ref, b_ref, ew1a_ref, ew1b_ref, eb1_ref,
               h_out_ref, a_ref, b_out_ref, x4_out_ref):
    mi = mi_ref[0]
    dx3 = xacc_ref[0, :, :3]
    h = h_ref[...]
    z = (jnp.dot(h, w1h_ref[...], preferred_element_type=jnp.float32)
         + jnp.dot(mi, w1m_ref[...],
                   preferred_element_type=jnp.float32) + nb1_ref[...])
    dh = jnp.dot(_silu(z), w2_ref[...],
                 preferred_element_type=jnp.float32) + nb2_ref[...]
    hn = h + dh
    mu = jnp.mean(hn, axis=1, keepdims=True)
    var = jnp.mean((hn - mu) ** 2, axis=1, keepdims=True)
    hln = (hn - mu) / jnp.sqrt(var + 1e-5) * g_ref[...] + b_ref[...]
    h_out_ref[...] = hln
    a_ref[...] = jnp.dot(hln, ew1a_ref[...],
                         preferred_element_type=jnp.float32) + eb1_ref[...]
    b_out_ref[...] = jnp.dot(hln, ew1b_ref[...],
                             preferred_element_type=jnp.float32)
    zero1 = jnp.zeros((dx3.shape[0], 1), jnp.float32)
    x4_out_ref[...] = x4_ref[...] + jnp.concatenate([dx3, zero1], axis=1)


def _node_update(h, accm, accx, x4, w1h, w1m, nb1, w2, nb2, g, b,
                 ew1a, ew1b, eb1):
    grid = (N // BN,)
    nb = lambda i: (i, 0)
    ab = lambda i: (i // 5, i % 5, 0)
    wb = lambda i: (0, 0)
    return pl.pallas_call(
        _node_body,
        grid=grid,
        in_specs=[
            pl.BlockSpec((BN, HID), nb),
            pl.BlockSpec((1, BN, HID), ab),
            pl.BlockSpec((1, BN, HID), ab),
            pl.BlockSpec((BN, 4), nb),
            pl.BlockSpec((HID, HID), wb),
            pl.BlockSpec((HID, HID), wb),
            pl.BlockSpec((1, HID), wb),
            pl.BlockSpec((HID, HID), wb),
            pl.BlockSpec((1, HID), wb),
            pl.BlockSpec((1, HID), wb),
            pl.BlockSpec((1, HID), wb),
            pl.BlockSpec((HID, HID), wb),
            pl.BlockSpec((HID, HID), wb),
            pl.BlockSpec((1, HID), wb),
        ],
        out_specs=[
            pl.BlockSpec((BN, HID), nb),
            pl.BlockSpec((BN, HID), nb),
            pl.BlockSpec((BN, HID), nb),
            pl.BlockSpec((BN, 4), nb),
        ],
        out_shape=[
            jax.ShapeDtypeStruct((N, HID), jnp.float32),
            jax.ShapeDtypeStruct((NP, HID), jnp.float32),
            jax.ShapeDtypeStruct((NP, HID), jnp.float32),
            jax.ShapeDtypeStruct((NP, 4), jnp.float32),
        ],
    )(h, accm, accx, x4, w1h, w1m, nb1, w2, nb2, g, b, ew1a, ew1b, eb1)


def _pro_body(h_ref, ew_ref, te_ref, ew1a_ref, ew1b_ref, eb1_ref,
              h_out_ref, a_ref, b_out_ref):
    h0 = (jnp.dot(h_ref[...], ew_ref[...], preferred_element_type=jnp.float32)
          + te_ref[...])
    h_out_ref[...] = h0
    a_ref[...] = jnp.dot(h0, ew1a_ref[...],
                         preferred_element_type=jnp.float32) + eb1_ref[...]
    b_out_ref[...] = jnp.dot(h0, ew1b_ref[...],
                             preferred_element_type=jnp.float32)


def _prologue(h, ew, te_row, ew1a, ew1b, eb1):
    grid = (N // BN,)
    nb = lambda i: (i, 0)
    wb = lambda i: (0, 0)
    return pl.pallas_call(
        _pro_body,
        grid=grid,
        in_specs=[
            pl.BlockSpec((BN, HID), nb),
            pl.BlockSpec((HID, HID), wb),
            pl.BlockSpec((1, HID), wb),
            pl.BlockSpec((HID, HID), wb),
            pl.BlockSpec((HID, HID), wb),
            pl.BlockSpec((1, HID), wb),
        ],
        out_specs=[pl.BlockSpec((BN, HID), nb)] * 3,
        out_shape=[
            jax.ShapeDtypeStruct((N, HID), jnp.float32),
            jax.ShapeDtypeStruct((NP, HID), jnp.float32),
            jax.ShapeDtypeStruct((NP, HID), jnp.float32),
        ],
    )(h, ew, te_row, ew1a, ew1b, eb1)


def _out_body(h_ref, w1_ref, b1_ref, w2_ref, b2_ref, o_ref):
    z = _silu(jnp.dot(h_ref[...], w1_ref[...],
                      preferred_element_type=jnp.float32) + b1_ref[...])
    o_ref[...] = jnp.dot(z, w2_ref[...],
                         preferred_element_type=jnp.float32) + b2_ref[...]


def _out_mlp(h, w1, b1, w2, b2):
    grid = (N // BN,)
    nb = lambda i: (i, 0)
    wb = lambda i: (0, 0)
    return pl.pallas_call(
        _out_body,
        grid=grid,
        in_specs=[
            pl.BlockSpec((BN, HID), nb),
            pl.BlockSpec((HID, HID), wb),
            pl.BlockSpec((1, HID), wb),
            pl.BlockSpec((HID, HID), wb),
            pl.BlockSpec((1, HID), wb),
        ],
        out_specs=pl.BlockSpec((BN, HID), nb),
        out_shape=jax.ShapeDtypeStruct((N, HID), jnp.float32),
    )(h, w1, b1, w2, b2)


def _sinusoidal(t, dim):
    half = dim // 2
    emb = math.log(10000.0) / (half - 1)
    emb = jnp.exp(jnp.arange(half, dtype=jnp.float32) * -emb)
    e = t[:, None] * emb[None, :]
    return jnp.concatenate([jnp.sin(e), jnp.cos(e)], axis=-1)


def kernel(h, x, params, edge_index, edge_attr, t):
    p = params
    row, col = edge_index[0], edge_index[1]

    te = _sinusoidal(t, TDIM)
    te = _silu(te @ p['time_w1'] + p['time_b1'])
    te = te @ p['time_w2'] + p['time_b2']
    te_row = te[0:1]

    padv = jnp.full((EP - E,), N, jnp.int32)
    ridx = jnp.concatenate([row, padv]).reshape(NIDX, 128)
    cidx = jnp.concatenate([col, padv]).reshape(NIDX, 128)
    ea_pad = jnp.pad(edge_attr, ((0, EP - E), (0, 0)))
    zeros_acc = jnp.zeros((NPA, HID), jnp.float32)

    layers = p['layers']
    lp0 = layers[0]
    h_cur, A, B = _prologue(h, p['embed_w'],
                            te_row + p['embed_b'][None, :],
                            lp0['edge_w1'][:HID],
                            lp0['edge_w1'][HID:2 * HID],
                            lp0['edge_b1'][None, :])
    x4 = jnp.pad(x, ((0, NP - N), (0, 1)))
    for li, lp in enumerate(layers):
        g1, g2, xd = _sc_gather(A, B, x4.reshape(-1), ridx, cidx)
        xd = xd.reshape(EP, 4)
        w1t = lp['edge_w1'][2 * HID:]
        srec, cdrec = _edge_mlp(g1, g2, xd, ea_pad, w1t,
                                lp['edge_w2'], lp['edge_b2'][None, :],
                                lp['att_w'], lp['att_b'][None, :],
                                lp['coord_w1'], lp['coord_b1'][None, :],
                                lp['coord_w2'])
        accm, accx = _sc_scatter(srec, cdrec.reshape(-1), ridx, zeros_acc)
        if li + 1 < len(layers):
            lpn = layers[li + 1]
            ew1a = lpn['edge_w1'][:HID]
            ew1b = lpn['edge_w1'][HID:2 * HID]
            eb1 = lpn['edge_b1'][None, :]
        else:
            ew1a = lp['edge_w1'][:HID]
            ew1b = lp['edge_w1'][HID:2 * HID]
            eb1 = lp['edge_b1'][None, :]
        h_cur, A, B, x4 = _node_update(
            h_cur, accm, accx, x4,
            lp['node_w1'][:HID], lp['node_w1'][HID:], lp['node_b1'][None, :],
            lp['node_w2'], lp['node_b2'][None, :],
            lp['ln_g'][None, :], lp['ln_b'][None, :],
            ew1a, ew1b, eb1)

    h_out = _out_mlp(h_cur, p['out_w1'], p['out_b1'][None, :],
                     p['out_w2'], p['out_b2'][None, :])
    return h_out, x4[:N, :3] - x


# speedup vs baseline: 2.1838x; 1.0267x over previous
import math

import jax
import jax.numpy as jnp
from jax import lax
from jax.experimental import pallas as pl
from jax.experimental.pallas import tpu as pltpu
from jax.experimental.pallas import tpu_sc as plsc

N = 10000
E = 320000
HID = 128
EDIM = 16
TDIM = 64

NP = 10112
EP = 327680
NIDX = 2560
KW = 80
NPH = 5000
NPA = 5120
RT2 = 320
KT = 160

BE = 4096
BN = 1000


def _silu(v):
    return v * jax.nn.sigmoid(v)


def _sc_gather_body(a_hbm, b_hbm, x4_hbm, ridx_hbm, cidx_hbm,
                    g1_hbm, g2_hbm, xd_hbm,
                    ridx_v, cidx_v, xtab,
                    gbufa0, gbufb0, xdbuf0, gbufa1, gbufb1, xdbuf1,
                    sa0, sb0, sa1, sb1, swa0, swb0, swx0, swa1, swb1, swx1):
    c = lax.axis_index("c")
    s = lax.axis_index("s")
    w = s * 2 + c
    base = w * KW
    pltpu.sync_copy(x4_hbm, xtab)
    lanes = lax.iota(jnp.int32, 16)

    def xd_compute(rrow, crow, xdbuf):
        for g in range(8):
            r16 = rrow[pl.ds(g * 16, 16)] * 4
            c16 = crow[pl.ds(g * 16, 16)] * 4
            pos = (g * 16 + lanes) * 4
            rad = None
            for k in range(3):
                xr = plsc.load_gather(xtab, [r16 + k])
                xc = plsc.load_gather(xtab, [c16 + k])
                d = xr - xc
                plsc.store_scatter(xdbuf, [pos + k], d)
                rad = d * d if rad is None else rad + d * d
            plsc.store_scatter(xdbuf, [pos + 3], rad)

    def pair(jo, blkbase):
        jb = jo * 2
        cpa0 = pltpu.async_copy(a_hbm.at[ridx_v.at[jb]], gbufa0, sa0)
        cpb0 = pltpu.async_copy(b_hbm.at[cidx_v.at[jb]], gbufb0, sb0)
        cpa1 = pltpu.async_copy(a_hbm.at[ridx_v.at[jb + 1]], gbufa1, sa1)
        cpb1 = pltpu.async_copy(b_hbm.at[cidx_v.at[jb + 1]], gbufb1, sb1)
        xd_compute(ridx_v.at[jb], cidx_v.at[jb], xdbuf0)
        xd_compute(ridx_v.at[jb + 1], cidx_v.at[jb + 1], xdbuf1)
        ebase0 = (blkbase + jb) * 128
        ebase1 = (blkbase + jb + 1) * 128
        cpa0.wait()
        cpb0.wait()
        w10 = pltpu.async_copy(gbufa0, g1_hbm.at[pl.ds(ebase0, 128)], swa0)
        w20 = pltpu.async_copy(gbufb0, g2_hbm.at[pl.ds(ebase0, 128)], swb0)
        w30 = pltpu.async_copy(xdbuf0, xd_hbm.at[pl.ds(ebase0 * 4, 512)], swx0)
        cpa1.wait()
        cpb1.wait()
        w11 = pltpu.async_copy(gbufa1, g1_hbm.at[pl.ds(ebase1, 128)], swa1)
        w21 = pltpu.async_copy(gbufb1, g2_hbm.at[pl.ds(ebase1, 128)], swb1)
        w31 = pltpu.async_copy(xdbuf1, xd_hbm.at[pl.ds(ebase1 * 4, 512)], swx1)
        w10.wait()
        w20.wait()
        w30.wait()
        w11.wait()
        w21.wait()
        w31.wait()
        return blkbase

    def block(bo, carry):
        pltpu.sync_copy(ridx_hbm.at[pl.ds(base + bo * 16, 16)], ridx_v)
        pltpu.sync_copy(cidx_hbm.at[pl.ds(base + bo * 16, 16)], cidx_v)
        lax.fori_loop(0, 8, pair, base + bo * 16)
        return carry

    lax.fori_loop(0, KW // 16, block, 0)


def _sc_gather(a, b, x4, ridx, cidx):
    mesh = plsc.VectorSubcoreMesh(core_axis_name="c", subcore_axis_name="s")
    f = pl.kernel(
        _sc_gather_body,
        out_type=[
            jax.ShapeDtypeStruct((EP, 128), jnp.float32),
            jax.ShapeDtypeStruct((EP, 128), jnp.float32),
            jax.ShapeDtypeStruct((EP * 4,), jnp.float32),
        ],
        mesh=mesh,
        scratch_types=[
            pltpu.VMEM((16, 128), jnp.int32),
            pltpu.VMEM((16, 128), jnp.int32),
            pltpu.VMEM((NP * 4,), jnp.float32),
            pltpu.VMEM((128, 128), jnp.float32),
            pltpu.VMEM((128, 128), jnp.float32),
            pltpu.VMEM((512,), jnp.float32),
            pltpu.VMEM((128, 128), jnp.float32),
            pltpu.VMEM((128, 128), jnp.float32),
            pltpu.VMEM((512,), jnp.float32),
        ] + [pltpu.SemaphoreType.DMA] * 10,
        compiler_params=pltpu.CompilerParams(needs_layout_passes=False),
    )
    return f(a, b, x4, ridx, cidx)


def _sc_scatter_body(s_hbm, cd_hbm, ridx_hbm, z_hbm, out_m_hbm, out_x_hbm,
                     ridx_v, sbuf0, sxbuf0, cdbuf0, idxbuf0,
                     sbuf1, sxbuf1, cdbuf1, idxbuf1, accm, accx,
                     sl0, sc0, sl1, sc1, sam0, sax0, sam1, sax1):
    c = lax.axis_index("c")
    s = lax.axis_index("s")
    pltpu.sync_copy(z_hbm.at[pl.ds(s * RT2, RT2)],
                    accm.at[pl.ds(s * RT2, RT2)])
    pltpu.sync_copy(z_hbm.at[pl.ds(s * RT2, RT2)],
                    accx.at[pl.ds(s * RT2, RT2)])
    pltpu.sync_copy(z_hbm.at[pl.ds(0, 64)], sxbuf0)
    pltpu.sync_copy(z_hbm.at[pl.ds(0, 64)], sxbuf1)
    plsc.subcore_barrier()
    lanes = lax.iota(jnp.int32, 16)
    base_node = c * NPH

    def route(rrow, half, idxbuf):
        for g in range(4):
            r16 = rrow[pl.ds(half * 64 + g * 16, 16)]
            loc = r16 - base_node
            ok = (loc >= 0) & (loc < NPH)
            idxbuf[pl.ds(g * 16, 16)] = jnp.where(ok, loc, NPH)

    def expand(cdbuf, sxbuf):
        for g in range(4):
            pos = g * 16 + lanes
            for k in range(3):
                v = plsc.load_gather(cdbuf, [pos * 4 + k])
                plsc.store_scatter(sxbuf, [pos, jnp.full((16,), k, jnp.int32)],
                                   v)

    def inner(ji, jo):
        j = jo * 16 + ji
        ebase = (s * KT + j) * 128
        l0 = pltpu.async_copy(s_hbm.at[pl.ds(ebase, 64)], sbuf0, sl0)
        c0 = pltpu.async_copy(cd_hbm.at[pl.ds(ebase * 4, 256)], cdbuf0, sc0)
        l1 = pltpu.async_copy(s_hbm.at[pl.ds(ebase + 64, 64)], sbuf1, sl1)
        c1 = pltpu.async_copy(cd_hbm.at[pl.ds(ebase * 4 + 256, 256)],
                              cdbuf1, sc1)
        rrow = ridx_v.at[ji]
        route(rrow, 0, idxbuf0)
        route(rrow, 1, idxbuf1)
        l0.wait()
        c0.wait()
        expand(cdbuf0, sxbuf0)
        a0 = pltpu.async_copy(sbuf0, accm.at[idxbuf0], sam0, add=True)
        x0 = pltpu.async_copy(sxbuf0, accx.at[idxbuf0], sax0, add=True)
        l1.wait()
        c1.wait()
        expand(cdbuf1, sxbuf1)
        a1 = pltpu.async_copy(sbuf1, accm.at[idxbuf1], sam1, add=True)
        x1 = pltpu.async_copy(sxbuf1, accx.at[idxbuf1], sax1, add=True)
        a0.wait()
        x0.wait()
        a1.wait()
        x1.wait()
        return jo

    def outer(jo, carry):
        pltpu.sync_copy(ridx_hbm.at[pl.ds(s * KT + jo * 16, 16)], ridx_v)
        lax.fori_loop(0, 16, inner, jo)
        return carry

    lax.fori_loop(0, KT // 16, outer, 0)
    plsc.subcore_barrier()
    pltpu.sync_copy(accm.at[pl.ds(s * RT2, RT2)],
                    out_m_hbm.at[c, pl.ds(s * RT2, RT2)])
    pltpu.sync_copy(accx.at[pl.ds(s * RT2, RT2)],
                    out_x_hbm.at[c, pl.ds(s * RT2, RT2)])


def _sc_scatter(srec, cd, ridx, zeros):
    mesh = plsc.VectorSubcoreMesh(core_axis_name="c", subcore_axis_name="s")
    f = pl.kernel(
        _sc_scatter_body,
        out_type=[
            jax.ShapeDtypeStruct((2, NPA, 128), jnp.float32),
            jax.ShapeDtypeStruct((2, NPA, 128), jnp.float32),
        ],
        mesh=mesh,
        scratch_types=[
            pltpu.VMEM((16, 128), jnp.int32),
            pltpu.VMEM((64, 128), jnp.float32),
            pltpu.VMEM((64, 128), jnp.float32),
            pltpu.VMEM((256,), jnp.float32),
            pltpu.VMEM((64,), jnp.int32),
            pltpu.VMEM((64, 128), jnp.float32),
            pltpu.VMEM((64, 128), jnp.float32),
            pltpu.VMEM((256,), jnp.float32),
            pltpu.VMEM((64,), jnp.int32),
            pltpu.VMEM_SHARED((NPA, 128), jnp.float32),
            pltpu.VMEM_SHARED((NPA, 128), jnp.float32),
        ] + [pltpu.SemaphoreType.DMA] * 8,
        compiler_params=pltpu.CompilerParams(needs_layout_passes=False),
    )
    return f(srec, cd, ridx, zeros)


def _edge_body(g1_ref, g2_ref, xd_ref, ea_ref, w1t_ref, w2_ref, b2_ref,
               attw_ref, attb_ref, cw1_ref, cb1_ref, cw2_ref,
               sm_ref, cd_ref):
    ein = g1_ref[...] + g2_ref[...]
    xdb = xd_ref[...]
    radial = xdb[:, 3:4]
    ra = jnp.concatenate([radial, ea_ref[...]], axis=1)
    t1 = ein + jnp.dot(ra, w1t_ref[...],
                       preferred_element_type=jnp.float32)
    m1 = _silu(t1)
    m2 = _silu(jnp.dot(m1.astype(jnp.bfloat16),
                       w2_ref[...].astype(jnp.bfloat16),
                       preferred_element_type=jnp.float32) + b2_ref[...])
    att = jax.nn.sigmoid(jnp.dot(m2, attw_ref[...],
                                 preferred_element_type=jnp.float32)
                         + attb_ref[0, 0])
    m = m2 * att
    c1 = _silu(jnp.dot(m.astype(jnp.bfloat16),
                       cw1_ref[...].astype(jnp.bfloat16),
                       preferred_element_type=jnp.float32) + cb1_ref[...])
    cw = jnp.dot(c1, cw2_ref[...], preferred_element_type=jnp.float32)
    sm_ref[...] = m
    cd_ref[...] = xdb * (cw / jnp.sqrt(radial + 1e-8))


def _edge_mlp(g1, g2, xd, eattr, w1t, w2, b2, attw, attb, cw1, cb1, cw2):
    grid = (EP // BE,)
    eb = lambda i: (i, 0)
    wb = lambda i: (0, 0)
    return pl.pallas_call(
        _edge_body,
        grid=grid,
        in_specs=[
            pl.BlockSpec((BE, HID), eb),
            pl.BlockSpec((BE, HID), eb),
            pl.BlockSpec((BE, 4), eb),
            pl.BlockSpec((BE, EDIM), eb),
            pl.BlockSpec((1 + EDIM, HID), wb),
            pl.BlockSpec((HID, HID), wb),
            pl.BlockSpec((1, HID), wb),
            pl.BlockSpec((HID, 1), wb),
            pl.BlockSpec((1, 1), wb),
            pl.BlockSpec((HID, HID), wb),
            pl.BlockSpec((1, HID), wb),
            pl.BlockSpec((HID, 1), wb),
        ],
        out_specs=[
            pl.BlockSpec((BE, HID), eb),
            pl.BlockSpec((BE, 4), eb),
        ],
        out_shape=[
            jax.ShapeDtypeStruct((EP, HID), jnp.float32),
            jax.ShapeDtypeStruct((EP, 4), jnp.float32),
        ],
    )(g1, g2, xd, eattr, w1t, w2, b2, attw, attb, cw1, cb1, cw2)


def _node_body(h_ref, mi_ref, xacc_ref, x4_ref, w1h_ref, w1m_ref, nb1_ref,
               w2_ref, nb2_ref, g_ref, b_ref, ew1a_ref, ew1b_ref, eb1_ref,
               h_out_ref, a_ref, b_out_ref, x4_out_ref):
    mi = mi_ref[0]
    dx3 = xacc_ref[0, :, :3]
    h = h_ref[...]
    z = (jnp.dot(h, w1h_ref[...], preferred_element_type=jnp.float32)
         + jnp.dot(mi, w1m_ref[...],
                   preferred_element_type=jnp.float32) + nb1_ref[...])
    dh = jnp.dot(_silu(z), w2_ref[...],
                 preferred_element_type=jnp.float32) + nb2_ref[...]
    hn = h + dh
    mu = jnp.mean(hn, axis=1, keepdims=True)
    var = jnp.mean((hn - mu) ** 2, axis=1, keepdims=True)
    hln = (hn - mu) / jnp.sqrt(var + 1e-5) * g_ref[...] + b_ref[...]
    h_out_ref[...] = hln
    a_ref[...] = jnp.dot(hln, ew1a_ref[...],
                         preferred_element_type=jnp.float32) + eb1_ref[...]
    b_out_ref[...] = jnp.dot(hln, ew1b_ref[...],
                             preferred_element_type=jnp.float32)
    zero1 = jnp.zeros((dx3.shape[0], 1), jnp.float32)
    x4_out_ref[...] = x4_ref[...] + jnp.concatenate([dx3, zero1], axis=1)


def _node_update(h, accm, accx, x4, w1h, w1m, nb1, w2, nb2, g, b,
                 ew1a, ew1b, eb1):
    grid = (N // BN,)
    nb = lambda i: (i, 0)
    ab = lambda i: (i // 5, i % 5, 0)
    wb = lambda i: (0, 0)
    return pl.pallas_call(
        _node_body,
        grid=grid,
        in_specs=[
            pl.BlockSpec((BN, HID), nb),
            pl.BlockSpec((1, BN, HID), ab),
            pl.BlockSpec((1, BN, HID), ab),
            pl.BlockSpec((BN, 4), nb),
            pl.BlockSpec((HID, HID), wb),
            pl.BlockSpec((HID, HID), wb),
            pl.BlockSpec((1, HID), wb),
            pl.BlockSpec((HID, HID), wb),
            pl.BlockSpec((1, HID), wb),
            pl.BlockSpec((1, HID), wb),
            pl.BlockSpec((1, HID), wb),
            pl.BlockSpec((HID, HID), wb),
            pl.BlockSpec((HID, HID), wb),
            pl.BlockSpec((1, HID), wb),
        ],
        out_specs=[
            pl.BlockSpec((BN, HID), nb),
            pl.BlockSpec((BN, HID), nb),
            pl.BlockSpec((BN, HID), nb),
            pl.BlockSpec((BN, 4), nb),
        ],
        out_shape=[
            jax.ShapeDtypeStruct((N, HID), jnp.float32),
            jax.ShapeDtypeStruct((NP, HID), jnp.float32),
            jax.ShapeDtypeStruct((NP, HID), jnp.float32),
            jax.ShapeDtypeStruct((NP, 4), jnp.float32),
        ],
    )(h, accm, accx, x4, w1h, w1m, nb1, w2, nb2, g, b, ew1a, ew1b, eb1)


def _pro_body(h_ref, ew_ref, te_ref, ew1a_ref, ew1b_ref, eb1_ref,
              h_out_ref, a_ref, b_out_ref):
    h0 = (jnp.dot(h_ref[...], ew_ref[...], preferred_element_type=jnp.float32)
          + te_ref[...])
    h_out_ref[...] = h0
    a_ref[...] = jnp.dot(h0, ew1a_ref[...],
                         preferred_element_type=jnp.float32) + eb1_ref[...]
    b_out_ref[...] = jnp.dot(h0, ew1b_ref[...],
                             preferred_element_type=jnp.float32)


def _prologue(h, ew, te_row, ew1a, ew1b, eb1):
    grid = (N // BN,)
    nb = lambda i: (i, 0)
    wb = lambda i: (0, 0)
    return pl.pallas_call(
        _pro_body,
        grid=grid,
        in_specs=[
            pl.BlockSpec((BN, HID), nb),
            pl.BlockSpec((HID, HID), wb),
            pl.BlockSpec((1, HID), wb),
            pl.BlockSpec((HID, HID), wb),
            pl.BlockSpec((HID, HID), wb),
            pl.BlockSpec((1, HID), wb),
        ],
        out_specs=[pl.BlockSpec((BN, HID), nb)] * 3,
        out_shape=[
            jax.ShapeDtypeStruct((N, HID), jnp.float32),
            jax.ShapeDtypeStruct((NP, HID), jnp.float32),
            jax.ShapeDtypeStruct((NP, HID), jnp.float32),
        ],
    )(h, ew, te_row, ew1a, ew1b, eb1)


def _out_body(h_ref, w1_ref, b1_ref, w2_ref, b2_ref, o_ref):
    z = _silu(jnp.dot(h_ref[...], w1_ref[...],
                      preferred_element_type=jnp.float32) + b1_ref[...])
    o_ref[...] = jnp.dot(z, w2_ref[...],
                         preferred_element_type=jnp.float32) + b2_ref[...]


def _out_mlp(h, w1, b1, w2, b2):
    grid = (N // BN,)
    nb = lambda i: (i, 0)
    wb = lambda i: (0, 0)
    return pl.pallas_call(
        _out_body,
        grid=grid,
        in_specs=[
            pl.BlockSpec((BN, HID), nb),
            pl.BlockSpec((HID, HID), wb),
            pl.BlockSpec((1, HID), wb),
            pl.BlockSpec((HID, HID), wb),
            pl.BlockSpec((1, HID), wb),
        ],
        out_specs=pl.BlockSpec((BN, HID), nb),
        out_shape=jax.ShapeDtypeStruct((N, HID), jnp.float32),
    )(h, w1, b1, w2, b2)


def _sinusoidal(t, dim):
    half = dim // 2
    emb = math.log(10000.0) / (half - 1)
    emb = jnp.exp(jnp.arange(half, dtype=jnp.float32) * -emb)
    e = t[:, None] * emb[None, :]
    return jnp.concatenate([jnp.sin(e), jnp.cos(e)], axis=-1)


def kernel(h, x, params, edge_index, edge_attr, t):
    p = params
    row, col = edge_index[0], edge_index[1]

    te = _sinusoidal(t, TDIM)
    te = _silu(te @ p['time_w1'] + p['time_b1'])
    te = te @ p['time_w2'] + p['time_b2']
    te_row = te[0:1]

    padv = jnp.full((EP - E,), N, jnp.int32)
    ridx = jnp.concatenate([row, padv]).reshape(NIDX, 128)
    cidx = jnp.concatenate([col, padv]).reshape(NIDX, 128)
    ea_pad = jnp.pad(edge_attr, ((0, EP - E), (0, 0)))
    zeros_acc = jnp.zeros((NPA, HID), jnp.float32)

    layers = p['layers']
    lp0 = layers[0]
    h_cur, A, B = _prologue(h, p['embed_w'],
                            te_row + p['embed_b'][None, :],
                            lp0['edge_w1'][:HID],
                            lp0['edge_w1'][HID:2 * HID],
                            lp0['edge_b1'][None, :])
    x4 = jnp.pad(x, ((0, NP - N), (0, 1)))
    for li, lp in enumerate(layers):
        g1, g2, xd = _sc_gather(A, B, x4.reshape(-1), ridx, cidx)
        xd = xd.reshape(EP, 4)
        w1t = lp['edge_w1'][2 * HID:]
        srec, cdrec = _edge_mlp(g1, g2, xd, ea_pad, w1t,
                                lp['edge_w2'], lp['edge_b2'][None, :],
                                lp['att_w'], lp['att_b'][None, :],
                                lp['coord_w1'], lp['coord_b1'][None, :],
                                lp['coord_w2'])
        accm, accx = _sc_scatter(srec, cdrec.reshape(-1), ridx, zeros_acc)
        if li + 1 < len(layers):
            lpn = layers[li + 1]
            ew1a = lpn['edge_w1'][:HID]
            ew1b = lpn['edge_w1'][HID:2 * HID]
            eb1 = lpn['edge_b1'][None, :]
        else:
            ew1a = lp['edge_w1'][:HID]
            ew1b = lp['edge_w1'][HID:2 * HID]
            eb1 = lp['edge_b1'][None, :]
        h_cur, A, B, x4 = _node_update(
            h_cur, accm, accx, x4,
            lp['node_w1'][:HID], lp['node_w1'][HID:], lp['node_b1'][None, :],
            lp['node_w2'], lp['node_b2'][None, :],
            lp['ln_g'][None, :], lp['ln_b'][None, :],
            ew1a, ew1b, eb1)

    h_out = _out_mlp(h_cur, p['out_w1'], p['out_b1'][None, :],
                     p['out_w2'], p['out_b2'][None, :])
    return h_out, x4[:N, :3] - x


# speedup vs baseline: 2.4319x; 1.1136x over previous
import math

import jax
import jax.numpy as jnp
from jax import lax
from jax.experimental import pallas as pl
from jax.experimental.pallas import tpu as pltpu
from jax.experimental.pallas import tpu_sc as plsc

N = 10000
E = 320000
HID = 128
EDIM = 16
TDIM = 64

NP = 10112
EP = 327680
NIDX = 2560
KW = 40
EPH = EP // 2
NIDXH = NIDX // 2
NPH = 5000
NPA = 5120
RT2 = 320
KT = 160

BE = 4096
BN = 1000


def _silu(v):
    return v * jax.nn.sigmoid(v)


def _sc_gather_body(a_hbm, b_hbm, x4_hbm, ridx_hbm, cidx_hbm,
                    g1_hbm, g2_hbm, xd_hbm,
                    ridx_v, cidx_v, xtab,
                    gbufa0, gbufb0, xdbuf0, gbufa1, gbufb1, xdbuf1,
                    sa0, sb0, sa1, sb1, swa0, swb0, swx0, swa1, swb1, swx1):
    c = lax.axis_index("c")
    s = lax.axis_index("s")
    w = s * 2 + c
    base = w * KW
    pltpu.sync_copy(x4_hbm, xtab)
    lanes = lax.iota(jnp.int32, 16)

    def xd_compute(rrow, crow, xdbuf):
        for g in range(8):
            r16 = rrow[pl.ds(g * 16, 16)] * 4
            c16 = crow[pl.ds(g * 16, 16)] * 4
            pos = (g * 16 + lanes) * 4
            rad = None
            for k in range(3):
                xr = plsc.load_gather(xtab, [r16 + k])
                xc = plsc.load_gather(xtab, [c16 + k])
                d = xr - xc
                plsc.store_scatter(xdbuf, [pos + k], d)
                rad = d * d if rad is None else rad + d * d
            plsc.store_scatter(xdbuf, [pos + 3], rad)

    def pair(jo, blkbase):
        jb = jo * 2
        cpa0 = pltpu.async_copy(a_hbm.at[ridx_v.at[jb]], gbufa0, sa0)
        cpb0 = pltpu.async_copy(b_hbm.at[cidx_v.at[jb]], gbufb0, sb0)
        cpa1 = pltpu.async_copy(a_hbm.at[ridx_v.at[jb + 1]], gbufa1, sa1)
        cpb1 = pltpu.async_copy(b_hbm.at[cidx_v.at[jb + 1]], gbufb1, sb1)
        xd_compute(ridx_v.at[jb], cidx_v.at[jb], xdbuf0)
        xd_compute(ridx_v.at[jb + 1], cidx_v.at[jb + 1], xdbuf1)
        ebase0 = (blkbase + jb) * 128
        ebase1 = (blkbase + jb + 1) * 128
        cpa0.wait()
        cpb0.wait()
        w10 = pltpu.async_copy(gbufa0, g1_hbm.at[pl.ds(ebase0, 128)], swa0)
        w20 = pltpu.async_copy(gbufb0, g2_hbm.at[pl.ds(ebase0, 128)], swb0)
        w30 = pltpu.async_copy(xdbuf0, xd_hbm.at[pl.ds(ebase0 * 4, 512)], swx0)
        cpa1.wait()
        cpb1.wait()
        w11 = pltpu.async_copy(gbufa1, g1_hbm.at[pl.ds(ebase1, 128)], swa1)
        w21 = pltpu.async_copy(gbufb1, g2_hbm.at[pl.ds(ebase1, 128)], swb1)
        w31 = pltpu.async_copy(xdbuf1, xd_hbm.at[pl.ds(ebase1 * 4, 512)], swx1)
        w10.wait()
        w20.wait()
        w30.wait()
        w11.wait()
        w21.wait()
        w31.wait()
        return blkbase

    def block(bo, carry):
        pltpu.sync_copy(ridx_hbm.at[pl.ds(base + bo * 8, 8)], ridx_v)
        pltpu.sync_copy(cidx_hbm.at[pl.ds(base + bo * 8, 8)], cidx_v)
        lax.fori_loop(0, 4, pair, base + bo * 8)
        return carry

    lax.fori_loop(0, KW // 8, block, 0)


def _sc_gather(a, b, x4, ridx, cidx):
    mesh = plsc.VectorSubcoreMesh(core_axis_name="c", subcore_axis_name="s")
    f = pl.kernel(
        _sc_gather_body,
        out_type=[
            jax.ShapeDtypeStruct((EPH, 128), jnp.float32),
            jax.ShapeDtypeStruct((EPH, 128), jnp.float32),
            jax.ShapeDtypeStruct((EPH * 4,), jnp.float32),
        ],
        mesh=mesh,
        scratch_types=[
            pltpu.VMEM((8, 128), jnp.int32),
            pltpu.VMEM((8, 128), jnp.int32),
            pltpu.VMEM((NP * 4,), jnp.float32),
            pltpu.VMEM((128, 128), jnp.float32),
            pltpu.VMEM((128, 128), jnp.float32),
            pltpu.VMEM((512,), jnp.float32),
            pltpu.VMEM((128, 128), jnp.float32),
            pltpu.VMEM((128, 128), jnp.float32),
            pltpu.VMEM((512,), jnp.float32),
        ] + [pltpu.SemaphoreType.DMA] * 10,
        compiler_params=pltpu.CompilerParams(needs_layout_passes=False),
    )
    return f(a, b, x4, ridx, cidx)


def _sc_scatter_body(s0_hbm, cd0_hbm, s1_hbm, cd1_hbm, ridx_hbm, z_hbm,
                     out_m_hbm, out_x_hbm,
                     ridx_v, sbuf0, sxbuf0, cdbuf0, idxbuf0,
                     sbuf1, sxbuf1, cdbuf1, idxbuf1, accm, accx,
                     sl0, sc0, sl1, sc1, sam0, sax0, sam1, sax1):
    c = lax.axis_index("c")
    s = lax.axis_index("s")
    pltpu.sync_copy(z_hbm.at[pl.ds(s * RT2, RT2)],
                    accm.at[pl.ds(s * RT2, RT2)])
    pltpu.sync_copy(z_hbm.at[pl.ds(s * RT2, RT2)],
                    accx.at[pl.ds(s * RT2, RT2)])
    pltpu.sync_copy(z_hbm.at[pl.ds(0, 64)], sxbuf0)
    pltpu.sync_copy(z_hbm.at[pl.ds(0, 64)], sxbuf1)
    plsc.subcore_barrier()
    lanes = lax.iota(jnp.int32, 16)
    base_node = c * NPH

    def route(rrow, half, idxbuf):
        for g in range(4):
            r16 = rrow[pl.ds(half * 64 + g * 16, 16)]
            loc = r16 - base_node
            ok = (loc >= 0) & (loc < NPH)
            idxbuf[pl.ds(g * 16, 16)] = jnp.where(ok, loc, NPH)

    def expand(cdbuf, sxbuf):
        for g in range(4):
            pos = g * 16 + lanes
            for k in range(3):
                v = plsc.load_gather(cdbuf, [pos * 4 + k])
                plsc.store_scatter(sxbuf, [pos, jnp.full((16,), k, jnp.int32)],
                                   v)

    def make_inner(s_hbm, cd_hbm, half):
        def inner(ji, jo):
            jl = s * 80 + jo * 16 + ji
            ebase = jl * 128
            l0 = pltpu.async_copy(s_hbm.at[pl.ds(ebase, 64)], sbuf0, sl0)
            c0 = pltpu.async_copy(cd_hbm.at[pl.ds(ebase * 4, 256)],
                                  cdbuf0, sc0)
            l1 = pltpu.async_copy(s_hbm.at[pl.ds(ebase + 64, 64)], sbuf1, sl1)
            c1 = pltpu.async_copy(cd_hbm.at[pl.ds(ebase * 4 + 256, 256)],
                                  cdbuf1, sc1)
            rrow = ridx_v.at[ji]
            route(rrow, 0, idxbuf0)
            route(rrow, 1, idxbuf1)
            l0.wait()
            c0.wait()
            expand(cdbuf0, sxbuf0)
            a0 = pltpu.async_copy(sbuf0, accm.at[idxbuf0], sam0, add=True)
            x0 = pltpu.async_copy(sxbuf0, accx.at[idxbuf0], sax0, add=True)
            l1.wait()
            c1.wait()
            expand(cdbuf1, sxbuf1)
            a1 = pltpu.async_copy(sbuf1, accm.at[idxbuf1], sam1, add=True)
            x1 = pltpu.async_copy(sxbuf1, accx.at[idxbuf1], sax1, add=True)
            a0.wait()
            x0.wait()
            a1.wait()
            x1.wait()
            return jo

        def outer(jo, carry):
            pltpu.sync_copy(
                ridx_hbm.at[pl.ds(half * NIDXH + s * 80 + jo * 16, 16)],
                ridx_v)
            lax.fori_loop(0, 16, inner, jo)
            return carry

        return outer

    lax.fori_loop(0, 5, make_inner(s0_hbm, cd0_hbm, 0), 0)
    lax.fori_loop(0, 5, make_inner(s1_hbm, cd1_hbm, 1), 0)
    plsc.subcore_barrier()
    pltpu.sync_copy(accm.at[pl.ds(s * RT2, RT2)],
                    out_m_hbm.at[c, pl.ds(s * RT2, RT2)])
    pltpu.sync_copy(accx.at[pl.ds(s * RT2, RT2)],
                    out_x_hbm.at[c, pl.ds(s * RT2, RT2)])


def _sc_scatter(s0, cd0, s1, cd1, ridx, zeros):
    mesh = plsc.VectorSubcoreMesh(core_axis_name="c", subcore_axis_name="s")
    f = pl.kernel(
        _sc_scatter_body,
        out_type=[
            jax.ShapeDtypeStruct((2, NPA, 128), jnp.float32),
            jax.ShapeDtypeStruct((2, NPA, 128), jnp.float32),
        ],
        mesh=mesh,
        scratch_types=[
            pltpu.VMEM((16, 128), jnp.int32),
            pltpu.VMEM((64, 128), jnp.float32),
            pltpu.VMEM((64, 128), jnp.float32),
            pltpu.VMEM((256,), jnp.float32),
            pltpu.VMEM((64,), jnp.int32),
            pltpu.VMEM((64, 128), jnp.float32),
            pltpu.VMEM((64, 128), jnp.float32),
            pltpu.VMEM((256,), jnp.float32),
            pltpu.VMEM((64,), jnp.int32),
            pltpu.VMEM_SHARED((NPA, 128), jnp.float32),
            pltpu.VMEM_SHARED((NPA, 128), jnp.float32),
        ] + [pltpu.SemaphoreType.DMA] * 8,
        compiler_params=pltpu.CompilerParams(needs_layout_passes=False),
    )
    return f(s0, cd0, s1, cd1, ridx, zeros)


def _edge_body(g1_ref, g2_ref, xd_ref, ea_ref, w1t_ref, w2_ref, b2_ref,
               attw_ref, attb_ref, cw1_ref, cb1_ref, cw2_ref,
               sm_ref, cd_ref):
    ein = g1_ref[...] + g2_ref[...]
    xdb = xd_ref[...]
    radial = xdb[:, 3:4]
    ra = jnp.concatenate([radial, ea_ref[...]], axis=1)
    t1 = ein + jnp.dot(ra, w1t_ref[...],
                       preferred_element_type=jnp.float32)
    m1 = _silu(t1)
    m2 = _silu(jnp.dot(m1.astype(jnp.bfloat16),
                       w2_ref[...].astype(jnp.bfloat16),
                       preferred_element_type=jnp.float32) + b2_ref[...])
    att = jax.nn.sigmoid(jnp.dot(m2, attw_ref[...],
                                 preferred_element_type=jnp.float32)
                         + attb_ref[0, 0])
    m = m2 * att
    c1 = _silu(jnp.dot(m.astype(jnp.bfloat16),
                       cw1_ref[...].astype(jnp.bfloat16),
                       preferred_element_type=jnp.float32) + cb1_ref[...])
    cw = jnp.dot(c1, cw2_ref[...], preferred_element_type=jnp.float32)
    sm_ref[...] = m
    cd_ref[...] = xdb * (cw / jnp.sqrt(radial + 1e-8))


def _edge_mlp(g1, g2, xd, eattr, w1t, w2, b2, attw, attb, cw1, cb1, cw2):
    grid = (EPH // BE,)
    eb = lambda i: (i, 0)
    wb = lambda i: (0, 0)
    return pl.pallas_call(
        _edge_body,
        grid=grid,
        in_specs=[
            pl.BlockSpec((BE, HID), eb),
            pl.BlockSpec((BE, HID), eb),
            pl.BlockSpec((BE, 4), eb),
            pl.BlockSpec((BE, EDIM), eb),
            pl.BlockSpec((1 + EDIM, HID), wb),
            pl.BlockSpec((HID, HID), wb),
            pl.BlockSpec((1, HID), wb),
            pl.BlockSpec((HID, 1), wb),
            pl.BlockSpec((1, 1), wb),
            pl.BlockSpec((HID, HID), wb),
            pl.BlockSpec((1, HID), wb),
            pl.BlockSpec((HID, 1), wb),
        ],
        out_specs=[
            pl.BlockSpec((BE, HID), eb),
            pl.BlockSpec((BE, 4), eb),
        ],
        out_shape=[
            jax.ShapeDtypeStruct((EPH, HID), jnp.float32),
            jax.ShapeDtypeStruct((EPH, 4), jnp.float32),
        ],
    )(g1, g2, xd, eattr, w1t, w2, b2, attw, attb, cw1, cb1, cw2)


def _node_body(h_ref, mi_ref, xacc_ref, x4_ref, w1h_ref, w1m_ref, nb1_ref,
               w2_ref, nb2_ref, g_ref, b_ref, ew1a_ref, ew1b_ref, eb1_ref,
               h_out_ref, a_ref, b_out_ref, x4_out_ref):
    mi = mi_ref[0]
    dx3 = xacc_ref[0, :, :3]
    h = h_ref[...]
    z = (jnp.dot(h, w1h_ref[...], preferred_element_type=jnp.float32)
         + jnp.dot(mi, w1m_ref[...],
                   preferred_element_type=jnp.float32) + nb1_ref[...])
    dh = jnp.dot(_silu(z), w2_ref[...],
                 preferred_element_type=jnp.float32) + nb2_ref[...]
    hn = h + dh
    mu = jnp.mean(hn, axis=1, keepdims=True)
    var = jnp.mean((hn - mu) ** 2, axis=1, keepdims=True)
    hln = (hn - mu) / jnp.sqrt(var + 1e-5) * g_ref[...] + b_ref[...]
    h_out_ref[...] = hln
    a_ref[...] = jnp.dot(hln, ew1a_ref[...],
                         preferred_element_type=jnp.float32) + eb1_ref[...]
    b_out_ref[...] = jnp.dot(hln, ew1b_ref[...],
                             preferred_element_type=jnp.float32)
    zero1 = jnp.zeros((dx3.shape[0], 1), jnp.float32)
    x4_out_ref[...] = x4_ref[...] + jnp.concatenate([dx3, zero1], axis=1)


def _node_update(h, accm, accx, x4, w1h, w1m, nb1, w2, nb2, g, b,
                 ew1a, ew1b, eb1):
    grid = (N // BN,)
    nb = lambda i: (i, 0)
    ab = lambda i: (i // 5, i % 5, 0)
    wb = lambda i: (0, 0)
    return pl.pallas_call(
        _node_body,
        grid=grid,
        in_specs=[
            pl.BlockSpec((BN, HID), nb),
            pl.BlockSpec((1, BN, HID), ab),
            pl.BlockSpec((1, BN, HID), ab),
            pl.BlockSpec((BN, 4), nb),
            pl.BlockSpec((HID, HID), wb),
            pl.BlockSpec((HID, HID), wb),
            pl.BlockSpec((1, HID), wb),
            pl.BlockSpec((HID, HID), wb),
            pl.BlockSpec((1, HID), wb),
            pl.BlockSpec((1, HID), wb),
            pl.BlockSpec((1, HID), wb),
            pl.BlockSpec((HID, HID), wb),
            pl.BlockSpec((HID, HID), wb),
            pl.BlockSpec((1, HID), wb),
        ],
        out_specs=[
            pl.BlockSpec((BN, HID), nb),
            pl.BlockSpec((BN, HID), nb),
            pl.BlockSpec((BN, HID), nb),
            pl.BlockSpec((BN, 4), nb),
        ],
        out_shape=[
            jax.ShapeDtypeStruct((N, HID), jnp.float32),
            jax.ShapeDtypeStruct((NP, HID), jnp.float32),
            jax.ShapeDtypeStruct((NP, HID), jnp.float32),
            jax.ShapeDtypeStruct((NP, 4), jnp.float32),
        ],
    )(h, accm, accx, x4, w1h, w1m, nb1, w2, nb2, g, b, ew1a, ew1b, eb1)


def _pro_body(h_ref, ew_ref, te_ref, ew1a_ref, ew1b_ref, eb1_ref,
              h_out_ref, a_ref, b_out_ref):
    h0 = (jnp.dot(h_ref[...], ew_ref[...], preferred_element_type=jnp.float32)
          + te_ref[...])
    h_out_ref[...] = h0
    a_ref[...] = jnp.dot(h0, ew1a_ref[...],
                         preferred_element_type=jnp.float32) + eb1_ref[...]
    b_out_ref[...] = jnp.dot(h0, ew1b_ref[...],
                             preferred_element_type=jnp.float32)


def _prologue(h, ew, te_row, ew1a, ew1b, eb1):
    grid = (N // BN,)
    nb = lambda i: (i, 0)
    wb = lambda i: (0, 0)
    return pl.pallas_call(
        _pro_body,
        grid=grid,
        in_specs=[
            pl.BlockSpec((BN, HID), nb),
            pl.BlockSpec((HID, HID), wb),
            pl.BlockSpec((1, HID), wb),
            pl.BlockSpec((HID, HID), wb),
            pl.BlockSpec((HID, HID), wb),
            pl.BlockSpec((1, HID), wb),
        ],
        out_specs=[pl.BlockSpec((BN, HID), nb)] * 3,
        out_shape=[
            jax.ShapeDtypeStruct((N, HID), jnp.float32),
            jax.ShapeDtypeStruct((NP, HID), jnp.float32),
            jax.ShapeDtypeStruct((NP, HID), jnp.float32),
        ],
    )(h, ew, te_row, ew1a, ew1b, eb1)


def _out_body(h_ref, w1_ref, b1_ref, w2_ref, b2_ref, o_ref):
    z = _silu(jnp.dot(h_ref[...], w1_ref[...],
                      preferred_element_type=jnp.float32) + b1_ref[...])
    o_ref[...] = jnp.dot(z, w2_ref[...],
                         preferred_element_type=jnp.float32) + b2_ref[...]


def _out_mlp(h, w1, b1, w2, b2):
    grid = (N // BN,)
    nb = lambda i: (i, 0)
    wb = lambda i: (0, 0)
    return pl.pallas_call(
        _out_body,
        grid=grid,
        in_specs=[
            pl.BlockSpec((BN, HID), nb),
            pl.BlockSpec((HID, HID), wb),
            pl.BlockSpec((1, HID), wb),
            pl.BlockSpec((HID, HID), wb),
            pl.BlockSpec((1, HID), wb),
        ],
        out_specs=pl.BlockSpec((BN, HID), nb),
        out_shape=jax.ShapeDtypeStruct((N, HID), jnp.float32),
    )(h, w1, b1, w2, b2)


def _sinusoidal(t, dim):
    half = dim // 2
    emb = math.log(10000.0) / (half - 1)
    emb = jnp.exp(jnp.arange(half, dtype=jnp.float32) * -emb)
    e = t[:, None] * emb[None, :]
    return jnp.concatenate([jnp.sin(e), jnp.cos(e)], axis=-1)


def kernel(h, x, params, edge_index, edge_attr, t):
    p = params
    row, col = edge_index[0], edge_index[1]

    te = _sinusoidal(t, TDIM)
    te = _silu(te @ p['time_w1'] + p['time_b1'])
    te = te @ p['time_w2'] + p['time_b2']
    te_row = te[0:1]

    padv = jnp.full((EP - E,), N, jnp.int32)
    ridx = jnp.concatenate([row, padv]).reshape(NIDX, 128)
    cidx = jnp.concatenate([col, padv]).reshape(NIDX, 128)
    ea_pad = jnp.pad(edge_attr, ((0, EP - E), (0, 0)))
    zeros_acc = jnp.zeros((NPA, HID), jnp.float32)

    layers = p['layers']
    lp0 = layers[0]
    h_cur, A, B = _prologue(h, p['embed_w'],
                            te_row + p['embed_b'][None, :],
                            lp0['edge_w1'][:HID],
                            lp0['edge_w1'][HID:2 * HID],
                            lp0['edge_b1'][None, :])
    x4 = jnp.pad(x, ((0, NP - N), (0, 1)))
    for li, lp in enumerate(layers):
        x4f = x4.reshape(-1)
        w1t = lp['edge_w1'][2 * HID:]
        ew = (lp['edge_w2'], lp['edge_b2'][None, :],
              lp['att_w'], lp['att_b'][None, :],
              lp['coord_w1'], lp['coord_b1'][None, :], lp['coord_w2'])
        g1a, g2a, xda = _sc_gather(A, B, x4f, ridx[:NIDXH], cidx[:NIDXH])
        g1b, g2b, xdb = _sc_gather(A, B, x4f, ridx[NIDXH:], cidx[NIDXH:])
        s0, cd0 = _edge_mlp(g1a, g2a, xda.reshape(EPH, 4), ea_pad[:EPH],
                            w1t, *ew)
        s1, cd1 = _edge_mlp(g1b, g2b, xdb.reshape(EPH, 4), ea_pad[EPH:],
                            w1t, *ew)
        accm, accx = _sc_scatter(s0, cd0.reshape(-1), s1, cd1.reshape(-1),
                                 ridx, zeros_acc)
        if li + 1 < len(layers):
            lpn = layers[li + 1]
            ew1a = lpn['edge_w1'][:HID]
            ew1b = lpn['edge_w1'][HID:2 * HID]
            eb1 = lpn['edge_b1'][None, :]
        else:
            ew1a = lp['edge_w1'][:HID]
            ew1b = lp['edge_w1'][HID:2 * HID]
            eb1 = lp['edge_b1'][None, :]
        h_cur, A, B, x4 = _node_update(
            h_cur, accm, accx, x4,
            lp['node_w1'][:HID], lp['node_w1'][HID:], lp['node_b1'][None, :],
            lp['node_w2'], lp['node_b2'][None, :],
            lp['ln_g'][None, :], lp['ln_b'][None, :],
            ew1a, ew1b, eb1)

    h_out = _out_mlp(h_cur, p['out_w1'], p['out_b1'][None, :],
                     p['out_w2'], p['out_b2'][None, :])
    return h_out, x4[:N, :3] - x
